# Initial kernel scaffold; baseline (speedup 1.0000x reference)
import jax
import jax.numpy as jnp
from jax.experimental import pallas as pl


def kernel(nodes, edge_index, edge_attr, M, alpha, pos_W, v0_W, v0_b, vR_W, vR_b, e0_W, e0_b, V1_W, V1_b, V2_W, V2_b, V3_W, V3_b, V4_W, V4_b, Ew_W, Ew_b, VBN_g, VBN_b, EBN_g, EBN_b):
    raise NotImplementedError("write your pallas kernel here")


# speedup vs baseline: 1.6170x; 1.6170x over previous
import functools

import jax
import jax.numpy as jnp
from jax import lax
from jax.experimental import pallas as pl
from jax.experimental.pallas import tpu as pltpu
from jax.experimental.pallas import tpu_sc as plsc

N_NODES = 50000
N_EDGES = 800000
UNITS = 48
DEPTH = 12
MP = 10000

NCORE = 2
NSUB = 16
CHUNK = 128
NCH = N_EDGES // CHUNK
HALF = N_NODES // 2
ROWS_PAD = 25088
RPS = ROWS_PAD // NSUB
RPS_LAST = HALF - RPS * (NSUB - 1)

f32 = jnp.float32


@functools.cache
def _mesh():
    return plsc.VectorSubcoreMesh(core_axis_name="c", subcore_axis_name="s")


def _sig(z):
    return 1.0 / (1.0 + jnp.exp(-z))


def _silu(z):
    return z * _sig(z)


def _local_idx(src_v, idx_v, core):
    base = core * HALF
    for kk in range(CHUNK // 16):
        sl = pl.ds(kk * 16, 16)
        v = src_v[sl] - base
        ok = (v >= 0) & (v < HALF)
        idx_v[sl] = jnp.where(ok, v, HALF)


@functools.cache
def _sc_cnt_fn():
    return functools.partial(
        pl.kernel,
        mesh=_mesh(),
        compiler_params=pltpu.CompilerParams(use_tc_tiling_on_sc=False),
        out_type=jax.ShapeDtypeStruct((N_NODES, UNITS), f32),
        scratch_types=[
            pltpu.VMEM((CHUNK,), jnp.int32),
            pltpu.VMEM((CHUNK,), jnp.int32),
            pltpu.VMEM((CHUNK, UNITS), f32),
            pltpu.VMEM_SHARED((ROWS_PAD, UNITS), f32),
        ],
    )(_sc_cnt_body)


def _sc_cnt_body(src_h, ones_h, zeros_h, out_h, src_v, idx_v, ones_v, aggS):
    core = lax.axis_index("c")
    sub = lax.axis_index("s")
    pltpu.sync_copy(zeros_h, aggS.at[pl.ds(sub * RPS, RPS)])
    pltpu.sync_copy(ones_h, ones_v)
    plsc.subcore_barrier()

    def body(j, carry):
        cid = j * NSUB + sub

        @pl.when(cid < NCH)
        def _():
            off = pl.multiple_of(cid * CHUNK, CHUNK)
            pltpu.sync_copy(src_h.at[pl.ds(off, CHUNK)], src_v)
            _local_idx(src_v, idx_v, core)
            pltpu.sync_copy(ones_v, aggS.at[idx_v], add=True)

        return carry

    lax.fori_loop(0, (NCH + NSUB - 1) // NSUB, body, 0)
    plsc.subcore_barrier()

    @pl.when(sub < NSUB - 1)
    def _():
        pltpu.sync_copy(
            aggS.at[pl.ds(sub * RPS, RPS)],
            out_h.at[pl.ds(core * HALF + sub * RPS, RPS)],
        )

    @pl.when(sub == NSUB - 1)
    def _():
        pltpu.sync_copy(
            aggS.at[pl.ds((NSUB - 1) * RPS, RPS_LAST)],
            out_h.at[pl.ds(core * HALF + (NSUB - 1) * RPS, RPS_LAST)],
        )


@functools.cache
def _sc_gsum_fn():
    return functools.partial(
        pl.kernel,
        mesh=_mesh(),
        compiler_params=pltpu.CompilerParams(use_tc_tiling_on_sc=False),
        out_type=jax.ShapeDtypeStruct((N_EDGES, UNITS), f32),
        scratch_types=[
            pltpu.VMEM((CHUNK,), jnp.int32),
            pltpu.VMEM((CHUNK,), jnp.int32),
            pltpu.VMEM((CHUNK, UNITS), f32),
            pltpu.VMEM((CHUNK, UNITS), f32),
            pltpu.SemaphoreType.DMA,
        ],
    )(_sc_gsum_body)


def _sc_gsum_body(src_h, dst_h, x3_h, x4_h, g_h, src_v, dst_v, r3, r4, sem):
    core = lax.axis_index("c")
    sub = lax.axis_index("s")
    wid = sub * NCORE + core

    def body(j, carry):
        cid = j * (NCORE * NSUB) + wid

        @pl.when(cid < NCH)
        def _():
            off = pl.multiple_of(cid * CHUNK, CHUNK)
            pltpu.sync_copy(src_h.at[pl.ds(off, CHUNK)], src_v)
            pltpu.sync_copy(dst_h.at[pl.ds(off, CHUNK)], dst_v)
            c1 = pltpu.async_copy(x3_h.at[src_v], r3, sem)
            c2 = pltpu.async_copy(x4_h.at[dst_v], r4, sem)
            c1.wait()
            c2.wait()

            def add_row(i, c):
                for kk in range(UNITS // 16):
                    sl = pl.ds(kk * 16, 16)
                    r3[i, sl] = r3[i, sl] + r4[i, sl]
                return c

            lax.fori_loop(0, CHUNK, add_row, 0, unroll=4)
            pltpu.sync_copy(r3, g_h.at[pl.ds(off, CHUNK)])

        return carry

    lax.fori_loop(0, (NCH + NCORE * NSUB - 1) // (NCORE * NSUB), body, 0)


@functools.cache
def _sc_agg_fn():
    return functools.partial(
        pl.kernel,
        mesh=_mesh(),
        compiler_params=pltpu.CompilerParams(use_tc_tiling_on_sc=False),
        out_type=jax.ShapeDtypeStruct((N_NODES, UNITS), f32),
        scratch_types=[
            pltpu.VMEM((CHUNK,), jnp.int32),
            pltpu.VMEM((CHUNK,), jnp.int32),
            pltpu.VMEM((CHUNK,), jnp.int32),
            pltpu.VMEM((CHUNK, UNITS), f32),
            pltpu.VMEM((CHUNK, UNITS), f32),
            pltpu.VMEM_SHARED((ROWS_PAD, UNITS), f32),
            pltpu.SemaphoreType.DMA,
        ],
    )(_sc_agg_body)


def _sc_agg_body(src_h, dst_h, x2_h, s_h, zeros_h, out_h,
                 src_v, dst_v, idx_v, r2, sv, aggS, sem):
    core = lax.axis_index("c")
    sub = lax.axis_index("s")
    pltpu.sync_copy(zeros_h, aggS.at[pl.ds(sub * RPS, RPS)])
    plsc.subcore_barrier()

    def body(j, carry):
        cid = j * NSUB + sub

        @pl.when(cid < NCH)
        def _():
            off = pl.multiple_of(cid * CHUNK, CHUNK)
            pltpu.sync_copy(src_h.at[pl.ds(off, CHUNK)], src_v)
            pltpu.sync_copy(dst_h.at[pl.ds(off, CHUNK)], dst_v)
            pltpu.async_copy(x2_h.at[dst_v], r2, sem).wait()
            pltpu.sync_copy(s_h.at[pl.ds(off, CHUNK)], sv)
            _local_idx(src_v, idx_v, core)

            def mul_row(i, c):
                for kk in range(UNITS // 16):
                    sl = pl.ds(kk * 16, 16)
                    r2[i, sl] = r2[i, sl] * sv[i, sl]
                return c

            lax.fori_loop(0, CHUNK, mul_row, 0, unroll=4)
            pltpu.sync_copy(r2, aggS.at[idx_v], add=True)

        return carry

    lax.fori_loop(0, (NCH + NSUB - 1) // NSUB, body, 0)
    plsc.subcore_barrier()

    @pl.when(sub < NSUB - 1)
    def _():
        pltpu.sync_copy(
            aggS.at[pl.ds(sub * RPS, RPS)],
            out_h.at[pl.ds(core * HALF + sub * RPS, RPS)],
        )

    @pl.when(sub == NSUB - 1)
    def _():
        pltpu.sync_copy(
            aggS.at[pl.ds((NSUB - 1) * RPS, RPS_LAST)],
            out_h.at[pl.ds(core * HALF + (NSUB - 1) * RPS, RPS_LAST)],
        )


EBLK = 8000
EGRID = N_EDGES // EBLK


def _ninit_body(nodes, v0W, v0b, vRW, vRb, posW, R, cosb, sinb, maskb, alpha,
                xo):
    i = pl.program_id(0)
    nb = nodes[...]
    xb = jnp.dot(nb, v0W[...], preferred_element_type=f32) + v0b[...]

    @pl.when(i < MP // 5000)
    def _():
        place = jnp.dot(nb, vRW[...], preferred_element_type=f32) + vRb[...]
        p2 = jnp.dot(place, R[...], preferred_element_type=f32)
        pe = place * cosb[...] + p2 * sinb[...]
        pe = jnp.dot(pe, posW[...], preferred_element_type=f32) * alpha[0, 0]
        xo[...] = _silu(xb + pe * maskb[...])

    @pl.when(i >= MP // 5000)
    def _():
        xo[...] = _silu(xb)


def _node_init(nodes, v0W, v0b, vRW, vRb, posW, R, cosb, sinb, maskb, alpha):
    blk = 5000
    pspec = pl.BlockSpec((blk, UNITS), lambda i: (jnp.minimum(i, 1), 0))
    return pl.pallas_call(
        _ninit_body,
        grid=(N_NODES // blk,),
        in_specs=[
            pl.BlockSpec((blk, 2), lambda i: (i, 0)),
            pl.BlockSpec((2, UNITS), lambda i: (0, 0)),
            pl.BlockSpec((1, UNITS), lambda i: (0, 0)),
            pl.BlockSpec((2, UNITS), lambda i: (0, 0)),
            pl.BlockSpec((1, UNITS), lambda i: (0, 0)),
            pl.BlockSpec((UNITS, UNITS), lambda i: (0, 0)),
            pl.BlockSpec((UNITS, UNITS), lambda i: (0, 0)),
            pspec,
            pspec,
            pspec,
            pl.BlockSpec((1, 1), lambda i: (0, 0)),
        ],
        out_specs=pl.BlockSpec((blk, UNITS), lambda i: (i, 0)),
        out_shape=jax.ShapeDtypeStruct((N_NODES, UNITS), f32),
    )(nodes, v0W, v0b, vRW, vRb, posW, R, cosb, sinb, maskb, alpha)


def _einit_body(ea, e0W, e0b, w_ref, s_ref):
    z = ea[...] * e0W[...] + e0b[...]
    wv = _silu(z)
    w_ref[...] = wv
    s_ref[...] = _sig(wv)


def _edge_init(ea, e0W, e0b):
    eb = jax.ShapeDtypeStruct((N_EDGES, UNITS), f32)
    return pl.pallas_call(
        _einit_body,
        grid=(EGRID,),
        in_specs=[
            pl.BlockSpec((EBLK, 1), lambda i: (i, 0)),
            pl.BlockSpec((1, UNITS), lambda i: (0, 0)),
            pl.BlockSpec((1, UNITS), lambda i: (0, 0)),
        ],
        out_specs=[
            pl.BlockSpec((EBLK, UNITS), lambda i: (i, 0)),
            pl.BlockSpec((EBLK, UNITS), lambda i: (i, 0)),
        ],
        out_shape=[eb, eb],
    )(ea, e0W, e0b)


NBLK = 5000
NGRID = N_NODES // NBLK


def _nmm_body(x, Wc, bc, o1, o2, o3, o4):
    acc = jnp.dot(x[...], Wc[...], preferred_element_type=f32) + bc[...]
    o1[...] = acc[:, 0 * UNITS:1 * UNITS]
    o2[...] = acc[:, 1 * UNITS:2 * UNITS]
    o3[...] = acc[:, 2 * UNITS:3 * UNITS]
    o4[...] = acc[:, 3 * UNITS:4 * UNITS]


def _node_mm4(x, Wc, bc):
    nb = jax.ShapeDtypeStruct((N_NODES, UNITS), f32)
    nspec = pl.BlockSpec((NBLK, UNITS), lambda i: (i, 0))
    return pl.pallas_call(
        _nmm_body,
        grid=(NGRID,),
        in_specs=[
            nspec,
            pl.BlockSpec((UNITS, 4 * UNITS), lambda i: (0, 0)),
            pl.BlockSpec((1, 4 * UNITS), lambda i: (0, 0)),
        ],
        out_specs=[nspec, nspec, nspec, nspec],
        out_shape=[nb, nb, nb, nb],
    )(x, Wc, bc)


def _emm_body(w0, g, Wr, br, t_ref, sums_ref):
    pid = pl.program_id(0)
    t = (jnp.dot(w0[...], Wr[...], preferred_element_type=f32) + br[...]
         + g[...])
    t_ref[...] = t

    @pl.when(pid == 0)
    def _():
        sums_ref[...] = jnp.zeros_like(sums_ref)

    sums_ref[0:1, :] += jnp.sum(t, axis=0, keepdims=True)
    sums_ref[1:2, :] += jnp.sum(t * t, axis=0, keepdims=True)


def _edge_mm_stats(w0, g, Wr, br):
    return pl.pallas_call(
        _emm_body,
        grid=(EGRID,),
        in_specs=[
            pl.BlockSpec((EBLK, UNITS), lambda i: (i, 0)),
            pl.BlockSpec((EBLK, UNITS), lambda i: (i, 0)),
            pl.BlockSpec((UNITS, UNITS), lambda i: (0, 0)),
            pl.BlockSpec((1, UNITS), lambda i: (0, 0)),
        ],
        out_specs=[
            pl.BlockSpec((EBLK, UNITS), lambda i: (i, 0)),
            pl.BlockSpec((2, UNITS), lambda i: (0, 0)),
        ],
        out_shape=[
            jax.ShapeDtypeStruct((N_EDGES, UNITS), f32),
            jax.ShapeDtypeStruct((2, UNITS), f32),
        ],
    )(w0, g, Wr, br)


def _nstat_body(x1, agg, cnt, t_ref, sums_ref):
    pid = pl.program_id(0)
    cn = cnt[...]
    a = jnp.where(cn > 0, agg[...] / jnp.maximum(cn, 1.0), 0.0)
    t = x1[...] + a
    t_ref[...] = t

    @pl.when(pid == 0)
    def _():
        sums_ref[...] = jnp.zeros_like(sums_ref)

    sums_ref[0:1, :] += jnp.sum(t, axis=0, keepdims=True)
    sums_ref[1:2, :] += jnp.sum(t * t, axis=0, keepdims=True)


def _napply_body(t, x0, sums, gm, bt, xo):
    inv_n = 1.0 / N_NODES
    mu = sums[0:1, :] * inv_n
    var = sums[1:2, :] * inv_n - mu * mu
    y = (t[...] - mu) * lax.rsqrt(var + 1e-5) * gm[...] + bt[...]
    xo[...] = x0[...] + _silu(y)


def _node_update(x1, agg, cnt, x0, gm, bt):
    nb = jax.ShapeDtypeStruct((N_NODES, UNITS), f32)
    nspec = pl.BlockSpec((NBLK, UNITS), lambda i: (i, 0))
    cspec = pl.BlockSpec((1, UNITS), lambda i: (0, 0))
    t, sums = pl.pallas_call(
        _nstat_body,
        grid=(NGRID,),
        in_specs=[nspec, nspec, nspec],
        out_specs=[nspec, pl.BlockSpec((2, UNITS), lambda i: (0, 0))],
        out_shape=[nb, jax.ShapeDtypeStruct((2, UNITS), f32)],
    )(x1, agg, cnt)
    return pl.pallas_call(
        _napply_body,
        grid=(NGRID,),
        in_specs=[nspec, nspec,
                  pl.BlockSpec((2, UNITS), lambda i: (0, 0)), cspec, cspec],
        out_specs=nspec,
        out_shape=nb,
    )(t, x0, sums, gm, bt)


def _eapply_body(t, w0, sums, gm, bt, w_ref, s_ref):
    inv_n = 1.0 / N_EDGES
    mu = sums[0:1, :] * inv_n
    var = sums[1:2, :] * inv_n - mu * mu
    y = (t[...] - mu) * lax.rsqrt(var + 1e-5) * gm[...] + bt[...]
    wn = w0[...] + _silu(y)
    w_ref[...] = wn
    s_ref[...] = _sig(wn)


def _edge_apply(t, w0, sums, gm, bt):
    eb = jax.ShapeDtypeStruct((N_EDGES, UNITS), f32)
    return pl.pallas_call(
        _eapply_body,
        grid=(EGRID,),
        in_specs=[
            pl.BlockSpec((EBLK, UNITS), lambda i: (i, 0)),
            pl.BlockSpec((EBLK, UNITS), lambda i: (i, 0)),
            pl.BlockSpec((2, UNITS), lambda i: (0, 0)),
            pl.BlockSpec((1, UNITS), lambda i: (0, 0)),
            pl.BlockSpec((1, UNITS), lambda i: (0, 0)),
        ],
        out_specs=[
            pl.BlockSpec((EBLK, UNITS), lambda i: (i, 0)),
            pl.BlockSpec((EBLK, UNITS), lambda i: (i, 0)),
        ],
        out_shape=[eb, eb],
    )(t, w0, sums, gm, bt)


def kernel(nodes, edge_index, edge_attr, M, alpha, pos_W, v0_W, v0_b, vR_W,
           vR_b, e0_W, e0_b, V1_W, V1_b, V2_W, V2_b, V3_W, V3_b, V4_W, V4_b,
           Ew_W, Ew_b, VBN_g, VBN_b, EBN_g, EBN_b):
    src = edge_index[0]
    dst = edge_index[1]

    ids = jnp.arange(UNITS // 2, dtype=f32)
    theta = jnp.power(1000.0, -2.0 * ids / UNITS)
    emb = jnp.arange(MP, dtype=f32)[:, None] * theta
    cosb = jnp.repeat(jnp.sin(emb), 2, axis=-1)
    sinb = jnp.repeat(jnp.cos(emb), 2, axis=-1)
    maskb = jnp.broadcast_to(
        (jnp.arange(MP) < M)[:, None], (MP, UNITS)).astype(f32)
    k2 = jnp.arange(UNITS // 2)
    R = (jnp.zeros((UNITS, UNITS), f32)
         .at[2 * k2 + 1, 2 * k2].set(-1.0)
         .at[2 * k2, 2 * k2 + 1].set(1.0))

    ones_h = jnp.ones((CHUNK, UNITS), f32)
    zeros_h = jnp.zeros((RPS, UNITS), f32)

    x = _node_init(nodes, v0_W, v0_b.reshape(1, -1), vR_W,
                   vR_b.reshape(1, -1), pos_W, R, cosb, sinb, maskb,
                   alpha.reshape(1, 1))
    w, s = _edge_init(edge_attr, e0_W, e0_b.reshape(1, -1))
    cntf = _sc_cnt_fn()(src, ones_h, zeros_h)

    for i in range(DEPTH):
        Wc = jnp.concatenate([V1_W[i], V2_W[i], V3_W[i], V4_W[i]], axis=1)
        bc = jnp.concatenate([V1_b[i], V2_b[i], V3_b[i], V4_b[i]]).reshape(1, -1)
        x1, x2, x3, x4 = _node_mm4(x, Wc, bc)
        g = _sc_gsum_fn()(src, dst, x3, x4)
        aggr = _sc_agg_fn()(src, dst, x2, s, zeros_h)
        t, sums = _edge_mm_stats(w, g, Ew_W[i], Ew_b[i].reshape(1, -1))
        x = _node_update(x1, aggr, cntf, x, VBN_g[i].reshape(1, -1),
                         VBN_b[i].reshape(1, -1))
        w, s = _edge_apply(t, w, sums, EBN_g[i].reshape(1, -1),
                           EBN_b[i].reshape(1, -1))
    return (x, w)


# speedup vs baseline: 1.7382x; 1.0749x over previous
import functools

import jax
import jax.numpy as jnp
from jax import lax
from jax.experimental import pallas as pl
from jax.experimental.pallas import tpu as pltpu
from jax.experimental.pallas import tpu_sc as plsc

N_NODES = 50000
N_EDGES = 800000
UNITS = 48
DEPTH = 12
MP = 10000

NCORE = 2
NSUB = 16
CHUNK = 128
NCH = N_EDGES // CHUNK
HALF = N_NODES // 2
ROWS_PAD = 25008
RPS = ROWS_PAD // NSUB
RPS_LAST = HALF - RPS * (NSUB - 1)

EPAD = 819200
NPAD = 50008
SUPE = 512
GPW_G = EPAD // CHUNK // (NCORE * NSUB)
NOUT_G = GPW_G * CHUNK // SUPE // 2
GPW_A = EPAD // CHUNK // NSUB
SUPE_A = 256
NOUT_A = GPW_A * CHUNK // SUPE_A // 2

f32 = jnp.float32


@functools.cache
def _mesh():
    return plsc.VectorSubcoreMesh(core_axis_name="c", subcore_axis_name="s")


def _sig(z):
    return 1.0 / (1.0 + jnp.exp(-z))


def _silu(z):
    return z * _sig(z)


def _local_idx(src_v, idx_v, core):
    base = core * HALF
    for kk in range(CHUNK // 16):
        sl = pl.ds(kk * 16, 16)
        v = src_v[sl] - base
        ok = (v >= 0) & (v < HALF)
        idx_v[sl] = jnp.where(ok, v, HALF)


@functools.cache
def _sc_cnt_fn():
    return functools.partial(
        pl.kernel,
        mesh=_mesh(),
        compiler_params=pltpu.CompilerParams(use_tc_tiling_on_sc=False),
        out_type=jax.ShapeDtypeStruct((N_NODES, UNITS), f32),
        scratch_types=[
            pltpu.VMEM((CHUNK,), jnp.int32),
            pltpu.VMEM((CHUNK,), jnp.int32),
            pltpu.VMEM((CHUNK, UNITS), f32),
            pltpu.VMEM_SHARED((ROWS_PAD, UNITS), f32),
        ],
    )(_sc_cnt_body)


def _sc_cnt_body(src_h, ones_h, zeros_h, out_h, src_v, idx_v, ones_v, aggS):
    core = lax.axis_index("c")
    sub = lax.axis_index("s")
    pltpu.sync_copy(zeros_h, aggS.at[pl.ds(sub * RPS, RPS)])
    pltpu.sync_copy(ones_h, ones_v)
    plsc.subcore_barrier()

    def body(j, carry):
        cid = j * NSUB + sub

        @pl.when(cid < NCH)
        def _():
            off = pl.multiple_of(cid * CHUNK, CHUNK)
            pltpu.sync_copy(src_h.at[pl.ds(off, CHUNK)], src_v)
            _local_idx(src_v, idx_v, core)
            pltpu.sync_copy(ones_v, aggS.at[idx_v], add=True)

        return carry

    lax.fori_loop(0, (NCH + NSUB - 1) // NSUB, body, 0)
    plsc.subcore_barrier()

    @pl.when(sub < NSUB - 1)
    def _():
        pltpu.sync_copy(
            aggS.at[pl.ds(sub * RPS, RPS)],
            out_h.at[pl.ds(core * HALF + sub * RPS, RPS)],
        )

    @pl.when(sub == NSUB - 1)
    def _():
        pltpu.sync_copy(
            aggS.at[pl.ds((NSUB - 1) * RPS, RPS_LAST)],
            out_h.at[pl.ds(core * HALF + (NSUB - 1) * RPS, RPS_LAST)],
        )


@functools.cache
def _sc_gsum_fn():
    return functools.partial(
        pl.kernel,
        mesh=_mesh(),
        compiler_params=pltpu.CompilerParams(use_tc_tiling_on_sc=False),
        out_type=jax.ShapeDtypeStruct((EPAD, UNITS), f32),
        scratch_types=[
            pltpu.VMEM((SUPE,), jnp.int32),
            pltpu.VMEM((SUPE,), jnp.int32),
            pltpu.VMEM((SUPE,), jnp.int32),
            pltpu.VMEM((SUPE,), jnp.int32),
            pltpu.VMEM((SUPE, UNITS), f32),
            pltpu.VMEM((SUPE, UNITS), f32),
            pltpu.VMEM((SUPE, UNITS), f32),
            pltpu.VMEM((SUPE, UNITS), f32),
            pltpu.SemaphoreType.DMA,
            pltpu.SemaphoreType.DMA,
            pltpu.SemaphoreType.DMA,
            pltpu.SemaphoreType.DMA,
            pltpu.SemaphoreType.DMA,
            pltpu.SemaphoreType.DMA,
        ],
    )(_sc_gsum_body)


def _sc_gsum_body(src_h, dst_h, x3_h, x4_h, g_h,
                  srcv0, srcv1, dstv0, dstv1, r30, r31, r40, r41,
                  semi0, semi1, semg0, semg1, semw0, semw1):
    core = lax.axis_index("c")
    sub = lax.axis_index("s")
    wid = sub * NCORE + core
    base = wid * GPW_G * CHUNK
    srcv = (srcv0, srcv1)
    dstv = (dstv0, dstv1)
    r3 = (r30, r31)
    r4 = (r40, r41)
    semi = (semi0, semi1)
    semg = (semg0, semg1)
    semw = (semw0, semw1)

    def body(it, carry):
        offs = [pl.multiple_of(base + (it * 2 + b) * SUPE, SUPE)
                for b in range(2)]
        icps = []
        for b in range(2):
            icps.append(pltpu.async_copy(
                src_h.at[pl.ds(offs[b], SUPE)], srcv[b], semi[b]))
            icps.append(pltpu.async_copy(
                dst_h.at[pl.ds(offs[b], SUPE)], dstv[b], semi[b]))
        gcps = [[], []]
        for b in range(2):
            @pl.when(it > 0)
            def _(b=b):
                pltpu.make_async_copy(
                    r3[b], g_h.at[pl.ds(offs[b], SUPE)], semw[b]).wait()

            icps[2 * b].wait()
            icps[2 * b + 1].wait()
            for k in range(SUPE // CHUNK):
                sl = pl.ds(k * CHUNK, CHUNK)
                gcps[b].append(pltpu.async_copy(
                    x3_h.at[srcv[b].at[sl]], r3[b].at[sl, :], semg[b]))
                gcps[b].append(pltpu.async_copy(
                    x4_h.at[dstv[b].at[sl]], r4[b].at[sl, :], semg[b]))
        for b in range(2):
            for cp in gcps[b]:
                cp.wait()

            def add_row(i, c, b=b):
                for kk in range(UNITS // 16):
                    sl = pl.ds(kk * 16, 16)
                    r3[b][i, sl] = r3[b][i, sl] + r4[b][i, sl]
                return c

            lax.fori_loop(0, SUPE, add_row, 0, unroll=4)
            pltpu.async_copy(r3[b], g_h.at[pl.ds(offs[b], SUPE)], semw[b])
        return carry

    lax.fori_loop(0, NOUT_G, body, 0)
    for b in range(2):
        pltpu.make_async_copy(r3[b], g_h.at[pl.ds(base, SUPE)], semw[b]).wait()


@functools.cache
def _sc_agg_fn():
    return functools.partial(
        pl.kernel,
        mesh=_mesh(),
        compiler_params=pltpu.CompilerParams(use_tc_tiling_on_sc=False),
        out_type=jax.ShapeDtypeStruct((N_NODES, UNITS), f32),
        scratch_types=[
            pltpu.VMEM((SUPE_A,), jnp.int32),
            pltpu.VMEM((SUPE_A,), jnp.int32),
            pltpu.VMEM((SUPE_A,), jnp.int32),
            pltpu.VMEM((SUPE_A,), jnp.int32),
            pltpu.VMEM((SUPE_A,), jnp.int32),
            pltpu.VMEM((SUPE_A,), jnp.int32),
            pltpu.VMEM((SUPE_A, UNITS), f32),
            pltpu.VMEM((SUPE_A, UNITS), f32),
            pltpu.VMEM((SUPE_A, UNITS), f32),
            pltpu.VMEM((SUPE_A, UNITS), f32),
            pltpu.VMEM_SHARED((ROWS_PAD, UNITS), f32),
            pltpu.SemaphoreType.DMA,
            pltpu.SemaphoreType.DMA,
            pltpu.SemaphoreType.DMA,
            pltpu.SemaphoreType.DMA,
        ],
    )(_sc_agg_body)


def _sc_agg_body(src_h, dst_h, x2_h, s_h, zeros_h, out_h,
                 srcv0, srcv1, dstv0, dstv1, lidx0, lidx1,
                 r20, r21, sv0, sv1, aggS,
                 semi0, semi1, semg0, semg1):
    core = lax.axis_index("c")
    sub = lax.axis_index("s")
    base = sub * GPW_A * CHUNK
    srcv = (srcv0, srcv1)
    dstv = (dstv0, dstv1)
    lidx = (lidx0, lidx1)
    r2 = (r20, r21)
    sv = (sv0, sv1)
    semi = (semi0, semi1)
    semg = (semg0, semg1)
    pltpu.sync_copy(zeros_h, aggS.at[pl.ds(sub * RPS, RPS)])
    plsc.subcore_barrier()

    def body(it, carry):
        offs = [pl.multiple_of(base + (it * 2 + b) * SUPE_A, SUPE_A)
                for b in range(2)]
        icps = []
        for b in range(2):
            icps.append(pltpu.async_copy(
                src_h.at[pl.ds(offs[b], SUPE_A)], srcv[b], semi[b]))
            icps.append(pltpu.async_copy(
                dst_h.at[pl.ds(offs[b], SUPE_A)], dstv[b], semi[b]))
            icps.append(pltpu.async_copy(
                s_h.at[pl.ds(offs[b], SUPE_A)], sv[b], semi[b]))
        gcps = [[], []]
        for b in range(2):
            icps[3 * b].wait()
            icps[3 * b + 1].wait()
            icps[3 * b + 2].wait()
            for k in range(SUPE_A // CHUNK):
                sl = pl.ds(k * CHUNK, CHUNK)
                gcps[b].append(pltpu.async_copy(
                    x2_h.at[dstv[b].at[sl]], r2[b].at[sl, :], semg[b]))
            for kk in range(SUPE_A // 16):
                sl = pl.ds(kk * 16, 16)
                v = srcv[b][sl] - core * HALF
                ok = (v >= 0) & (v < HALF)
                lidx[b][sl] = jnp.where(ok, v, HALF)
        for b in range(2):
            for cp in gcps[b]:
                cp.wait()

            def mul_row(i, c, b=b):
                for kk in range(UNITS // 16):
                    sl = pl.ds(kk * 16, 16)
                    r2[b][i, sl] = r2[b][i, sl] * sv[b][i, sl]
                return c

            lax.fori_loop(0, SUPE_A, mul_row, 0, unroll=4)
            pltpu.sync_copy(r2[b], aggS.at[lidx[b]], add=True)
        return carry

    lax.fori_loop(0, NOUT_A, body, 0)
    plsc.subcore_barrier()

    @pl.when(sub < NSUB - 1)
    def _():
        pltpu.sync_copy(
            aggS.at[pl.ds(sub * RPS, RPS)],
            out_h.at[pl.ds(core * HALF + sub * RPS, RPS)],
        )

    @pl.when(sub == NSUB - 1)
    def _():
        pltpu.sync_copy(
            aggS.at[pl.ds((NSUB - 1) * RPS, RPS_LAST)],
            out_h.at[pl.ds(core * HALF + (NSUB - 1) * RPS, RPS_LAST)],
        )


EBLK = 8000
EGRID = N_EDGES // EBLK


def _ninit_body(nodes, v0W, v0b, vRW, vRb, posW, R, cosb, sinb, maskb, alpha,
                xo):
    i = pl.program_id(0)
    nb = nodes[...]
    xb = jnp.dot(nb, v0W[...], preferred_element_type=f32) + v0b[...]

    @pl.when(i < MP // 5000)
    def _():
        place = jnp.dot(nb, vRW[...], preferred_element_type=f32) + vRb[...]
        p2 = jnp.dot(place, R[...], preferred_element_type=f32)
        pe = place * cosb[...] + p2 * sinb[...]
        pe = jnp.dot(pe, posW[...], preferred_element_type=f32) * alpha[0, 0]
        xo[...] = _silu(xb + pe * maskb[...])

    @pl.when(i >= MP // 5000)
    def _():
        xo[...] = _silu(xb)


def _node_init(nodes, v0W, v0b, vRW, vRb, posW, R, cosb, sinb, maskb, alpha):
    blk = 5000
    pspec = pl.BlockSpec((blk, UNITS), lambda i: (jnp.minimum(i, 1), 0))
    return pl.pallas_call(
        _ninit_body,
        grid=(N_NODES // blk,),
        in_specs=[
            pl.BlockSpec((blk, 2), lambda i: (i, 0)),
            pl.BlockSpec((2, UNITS), lambda i: (0, 0)),
            pl.BlockSpec((1, UNITS), lambda i: (0, 0)),
            pl.BlockSpec((2, UNITS), lambda i: (0, 0)),
            pl.BlockSpec((1, UNITS), lambda i: (0, 0)),
            pl.BlockSpec((UNITS, UNITS), lambda i: (0, 0)),
            pl.BlockSpec((UNITS, UNITS), lambda i: (0, 0)),
            pspec,
            pspec,
            pspec,
            pl.BlockSpec((1, 1), lambda i: (0, 0)),
        ],
        out_specs=pl.BlockSpec((blk, UNITS), lambda i: (i, 0)),
        out_shape=jax.ShapeDtypeStruct((N_NODES, UNITS), f32),
    )(nodes, v0W, v0b, vRW, vRb, posW, R, cosb, sinb, maskb, alpha)


def _einit_body(ea, e0W, e0b, w_ref, s_ref):
    z = ea[...] * e0W[...] + e0b[...]
    wv = _silu(z)
    w_ref[...] = wv
    s_ref[...] = _sig(wv)


def _edge_init(ea, e0W, e0b):
    return pl.pallas_call(
        _einit_body,
        grid=(EGRID,),
        in_specs=[
            pl.BlockSpec((EBLK, 1), lambda i: (i, 0)),
            pl.BlockSpec((1, UNITS), lambda i: (0, 0)),
            pl.BlockSpec((1, UNITS), lambda i: (0, 0)),
        ],
        out_specs=[
            pl.BlockSpec((EBLK, UNITS), lambda i: (i, 0)),
            pl.BlockSpec((EBLK, UNITS), lambda i: (i, 0)),
        ],
        out_shape=[jax.ShapeDtypeStruct((N_EDGES, UNITS), f32),
                   jax.ShapeDtypeStruct((EPAD, UNITS), f32)],
    )(ea, e0W, e0b)


NBLK = 5000
NGRID = N_NODES // NBLK


def _nmm_body(x, Wc, bc, o1, o2, o3, o4):
    acc = jnp.dot(x[...], Wc[...], preferred_element_type=f32) + bc[...]
    o1[...] = acc[:, 0 * UNITS:1 * UNITS]
    o2[...] = acc[:, 1 * UNITS:2 * UNITS]
    o3[...] = acc[:, 2 * UNITS:3 * UNITS]
    o4[...] = acc[:, 3 * UNITS:4 * UNITS]


def _node_mm4(x, Wc, bc):
    nb = jax.ShapeDtypeStruct((NPAD, UNITS), f32)
    nspec = pl.BlockSpec((NBLK, UNITS), lambda i: (i, 0))
    return pl.pallas_call(
        _nmm_body,
        grid=(NGRID,),
        in_specs=[
            nspec,
            pl.BlockSpec((UNITS, 4 * UNITS), lambda i: (0, 0)),
            pl.BlockSpec((1, 4 * UNITS), lambda i: (0, 0)),
        ],
        out_specs=[nspec, nspec, nspec, nspec],
        out_shape=[nb, nb, nb, nb],
    )(x, Wc, bc)


def _emm_body(w0, g, Wr, br, t_ref, sums_ref):
    pid = pl.program_id(0)
    t = (jnp.dot(w0[...], Wr[...], preferred_element_type=f32) + br[...]
         + g[...])
    t_ref[...] = t

    @pl.when(pid == 0)
    def _():
        sums_ref[...] = jnp.zeros_like(sums_ref)

    sums_ref[0:1, :] += jnp.sum(t, axis=0, keepdims=True)
    sums_ref[1:2, :] += jnp.sum(t * t, axis=0, keepdims=True)


def _edge_mm_stats(w0, g, Wr, br):
    return pl.pallas_call(
        _emm_body,
        grid=(EGRID,),
        in_specs=[
            pl.BlockSpec((EBLK, UNITS), lambda i: (i, 0)),
            pl.BlockSpec((EBLK, UNITS), lambda i: (i, 0)),
            pl.BlockSpec((UNITS, UNITS), lambda i: (0, 0)),
            pl.BlockSpec((1, UNITS), lambda i: (0, 0)),
        ],
        out_specs=[
            pl.BlockSpec((EBLK, UNITS), lambda i: (i, 0)),
            pl.BlockSpec((2, UNITS), lambda i: (0, 0)),
        ],
        out_shape=[
            jax.ShapeDtypeStruct((N_EDGES, UNITS), f32),
            jax.ShapeDtypeStruct((2, UNITS), f32),
        ],
    )(w0, g, Wr, br)


def _nstat_body(x1, agg, cnt, t_ref, sums_ref):
    pid = pl.program_id(0)
    cn = cnt[...]
    a = jnp.where(cn > 0, agg[...] / jnp.maximum(cn, 1.0), 0.0)
    t = x1[...] + a
    t_ref[...] = t

    @pl.when(pid == 0)
    def _():
        sums_ref[...] = jnp.zeros_like(sums_ref)

    sums_ref[0:1, :] += jnp.sum(t, axis=0, keepdims=True)
    sums_ref[1:2, :] += jnp.sum(t * t, axis=0, keepdims=True)


def _napply_body(t, x0, sums, gm, bt, xo):
    inv_n = 1.0 / N_NODES
    mu = sums[0:1, :] * inv_n
    var = sums[1:2, :] * inv_n - mu * mu
    y = (t[...] - mu) * lax.rsqrt(var + 1e-5) * gm[...] + bt[...]
    xo[...] = x0[...] + _silu(y)


def _node_update(x1, agg, cnt, x0, gm, bt):
    nb = jax.ShapeDtypeStruct((N_NODES, UNITS), f32)
    nspec = pl.BlockSpec((NBLK, UNITS), lambda i: (i, 0))
    cspec = pl.BlockSpec((1, UNITS), lambda i: (0, 0))
    t, sums = pl.pallas_call(
        _nstat_body,
        grid=(NGRID,),
        in_specs=[nspec, nspec, nspec],
        out_specs=[nspec, pl.BlockSpec((2, UNITS), lambda i: (0, 0))],
        out_shape=[nb, jax.ShapeDtypeStruct((2, UNITS), f32)],
    )(x1, agg, cnt)
    return pl.pallas_call(
        _napply_body,
        grid=(NGRID,),
        in_specs=[nspec, nspec,
                  pl.BlockSpec((2, UNITS), lambda i: (0, 0)), cspec, cspec],
        out_specs=nspec,
        out_shape=nb,
    )(t, x0, sums, gm, bt)


def _eapply_body(t, w0, sums, gm, bt, w_ref, s_ref):
    inv_n = 1.0 / N_EDGES
    mu = sums[0:1, :] * inv_n
    var = sums[1:2, :] * inv_n - mu * mu
    y = (t[...] - mu) * lax.rsqrt(var + 1e-5) * gm[...] + bt[...]
    wn = w0[...] + _silu(y)
    w_ref[...] = wn
    s_ref[...] = _sig(wn)


def _edge_apply(t, w0, sums, gm, bt):
    eb = jax.ShapeDtypeStruct((N_EDGES, UNITS), f32)
    sb = jax.ShapeDtypeStruct((EPAD, UNITS), f32)
    return pl.pallas_call(
        _eapply_body,
        grid=(EGRID,),
        in_specs=[
            pl.BlockSpec((EBLK, UNITS), lambda i: (i, 0)),
            pl.BlockSpec((EBLK, UNITS), lambda i: (i, 0)),
            pl.BlockSpec((2, UNITS), lambda i: (0, 0)),
            pl.BlockSpec((1, UNITS), lambda i: (0, 0)),
            pl.BlockSpec((1, UNITS), lambda i: (0, 0)),
        ],
        out_specs=[
            pl.BlockSpec((EBLK, UNITS), lambda i: (i, 0)),
            pl.BlockSpec((EBLK, UNITS), lambda i: (i, 0)),
        ],
        out_shape=[eb, sb],
    )(t, w0, sums, gm, bt)


def kernel(nodes, edge_index, edge_attr, M, alpha, pos_W, v0_W, v0_b, vR_W,
           vR_b, e0_W, e0_b, V1_W, V1_b, V2_W, V2_b, V3_W, V3_b, V4_W, V4_b,
           Ew_W, Ew_b, VBN_g, VBN_b, EBN_g, EBN_b):
    src = edge_index[0]
    dst = edge_index[1]
    npad = EPAD - N_EDGES
    srcp = jnp.concatenate(
        [src, jnp.full((npad,), N_NODES, dtype=jnp.int32)])
    dstp = jnp.concatenate([dst, jnp.zeros((npad,), dtype=jnp.int32)])

    ids = jnp.arange(UNITS // 2, dtype=f32)
    theta = jnp.power(1000.0, -2.0 * ids / UNITS)
    emb = jnp.arange(MP, dtype=f32)[:, None] * theta
    cosb = jnp.repeat(jnp.sin(emb), 2, axis=-1)
    sinb = jnp.repeat(jnp.cos(emb), 2, axis=-1)
    maskb = jnp.broadcast_to(
        (jnp.arange(MP) < M)[:, None], (MP, UNITS)).astype(f32)
    k2 = jnp.arange(UNITS // 2)
    R = (jnp.zeros((UNITS, UNITS), f32)
         .at[2 * k2 + 1, 2 * k2].set(-1.0)
         .at[2 * k2, 2 * k2 + 1].set(1.0))

    ones_h = jnp.ones((CHUNK, UNITS), f32)
    zeros_h = jnp.zeros((RPS, UNITS), f32)

    x = _node_init(nodes, v0_W, v0_b.reshape(1, -1), vR_W,
                   vR_b.reshape(1, -1), pos_W, R, cosb, sinb, maskb,
                   alpha.reshape(1, 1))
    w, s = _edge_init(edge_attr, e0_W, e0_b.reshape(1, -1))
    cntf = _sc_cnt_fn()(src, ones_h, zeros_h)

    for i in range(DEPTH):
        Wc = jnp.concatenate([V1_W[i], V2_W[i], V3_W[i], V4_W[i]], axis=1)
        bc = jnp.concatenate([V1_b[i], V2_b[i], V3_b[i], V4_b[i]]).reshape(1, -1)
        x1, x2, x3, x4 = _node_mm4(x, Wc, bc)
        g = _sc_gsum_fn()(srcp, dstp, x3, x4)
        aggr = _sc_agg_fn()(srcp, dstp, x2, s, zeros_h)
        t, sums = _edge_mm_stats(w, g, Ew_W[i], Ew_b[i].reshape(1, -1))
        x = _node_update(x1, aggr, cntf, x, VBN_g[i].reshape(1, -1),
                         VBN_b[i].reshape(1, -1))
        w, s = _edge_apply(t, w, sums, EBN_g[i].reshape(1, -1),
                           EBN_b[i].reshape(1, -1))
    return (x, w)


# speedup vs baseline: 1.7412x; 1.0017x over previous
import functools

import jax
import jax.numpy as jnp
from jax import lax
from jax.experimental import pallas as pl
from jax.experimental.pallas import tpu as pltpu
from jax.experimental.pallas import tpu_sc as plsc

N_NODES = 50000
N_EDGES = 800000
UNITS = 48
DEPTH = 12
MP = 10000

NCORE = 2
NSUB = 16
CHUNK = 128
NCH = N_EDGES // CHUNK
HALF = N_NODES // 2
ROWS_PAD = 25008
RPS = ROWS_PAD // NSUB
RPS_LAST = HALF - RPS * (NSUB - 1)

EPAD = 819200
NPAD = 50008
SUPE = 512
GPW_G = EPAD // CHUNK // (NCORE * NSUB)
NOUT_G = GPW_G * CHUNK // SUPE // 2
GPW_A = EPAD // CHUNK // NSUB
SUPE_A = 256
NOUT_A = GPW_A * CHUNK // SUPE_A // 2

f32 = jnp.float32


@functools.cache
def _mesh():
    return plsc.VectorSubcoreMesh(core_axis_name="c", subcore_axis_name="s")


def _sig(z):
    return 1.0 / (1.0 + jnp.exp(-z))


def _silu(z):
    return z * _sig(z)


SUPE_C = 512
NOUT_C = GPW_A * CHUNK // SUPE_C // 2


@functools.cache
def _sc_cnt_fn():
    return functools.partial(
        pl.kernel,
        mesh=_mesh(),
        compiler_params=pltpu.CompilerParams(use_tc_tiling_on_sc=False),
        out_type=jax.ShapeDtypeStruct((N_NODES, UNITS), f32),
        scratch_types=[
            pltpu.VMEM((SUPE_C,), jnp.int32),
            pltpu.VMEM((SUPE_C,), jnp.int32),
            pltpu.VMEM((SUPE_C,), jnp.int32),
            pltpu.VMEM((SUPE_C,), jnp.int32),
            pltpu.VMEM((SUPE_C, UNITS), f32),
            pltpu.VMEM_SHARED((ROWS_PAD, UNITS), f32),
            pltpu.SemaphoreType.DMA,
            pltpu.SemaphoreType.DMA,
        ],
    )(_sc_cnt_body)


def _sc_cnt_body(src_h, ones_h, zeros_h, out_h,
                 srcv0, srcv1, lidx0, lidx1, ones_v, aggS, semi0, semi1):
    core = lax.axis_index("c")
    sub = lax.axis_index("s")
    base = sub * GPW_A * CHUNK
    srcv = (srcv0, srcv1)
    lidx = (lidx0, lidx1)
    semi = (semi0, semi1)
    pltpu.sync_copy(zeros_h, aggS.at[pl.ds(sub * RPS, RPS)])
    pltpu.sync_copy(ones_h, ones_v)
    plsc.subcore_barrier()

    def body(it, carry):
        offs = [pl.multiple_of(base + (it * 2 + b) * SUPE_C, SUPE_C)
                for b in range(2)]
        icps = [pltpu.async_copy(
            src_h.at[pl.ds(offs[b], SUPE_C)], srcv[b], semi[b])
            for b in range(2)]
        for b in range(2):
            icps[b].wait()
            for kk in range(SUPE_C // 16):
                sl = pl.ds(kk * 16, 16)
                v = srcv[b][sl] - core * HALF
                ok = (v >= 0) & (v < HALF)
                lidx[b][sl] = jnp.where(ok, v, HALF)
            pltpu.sync_copy(ones_v, aggS.at[lidx[b]], add=True)
        return carry

    lax.fori_loop(0, NOUT_C, body, 0)
    plsc.subcore_barrier()

    @pl.when(sub < NSUB - 1)
    def _():
        pltpu.sync_copy(
            aggS.at[pl.ds(sub * RPS, RPS)],
            out_h.at[pl.ds(core * HALF + sub * RPS, RPS)],
        )

    @pl.when(sub == NSUB - 1)
    def _():
        pltpu.sync_copy(
            aggS.at[pl.ds((NSUB - 1) * RPS, RPS_LAST)],
            out_h.at[pl.ds(core * HALF + (NSUB - 1) * RPS, RPS_LAST)],
        )


@functools.cache
def _sc_gsum_fn():
    return functools.partial(
        pl.kernel,
        mesh=_mesh(),
        compiler_params=pltpu.CompilerParams(use_tc_tiling_on_sc=False),
        out_type=jax.ShapeDtypeStruct((EPAD, UNITS), f32),
        scratch_types=[
            pltpu.VMEM((SUPE,), jnp.int32),
            pltpu.VMEM((SUPE,), jnp.int32),
            pltpu.VMEM((SUPE,), jnp.int32),
            pltpu.VMEM((SUPE,), jnp.int32),
            pltpu.VMEM((SUPE, UNITS), f32),
            pltpu.VMEM((SUPE, UNITS), f32),
            pltpu.VMEM((SUPE, UNITS), f32),
            pltpu.VMEM((SUPE, UNITS), f32),
            pltpu.SemaphoreType.DMA,
            pltpu.SemaphoreType.DMA,
            pltpu.SemaphoreType.DMA,
            pltpu.SemaphoreType.DMA,
            pltpu.SemaphoreType.DMA,
            pltpu.SemaphoreType.DMA,
        ],
    )(_sc_gsum_body)


def _sc_gsum_body(src_h, dst_h, x3_h, x4_h, g_h,
                  srcv0, srcv1, dstv0, dstv1, r30, r31, r40, r41,
                  semi0, semi1, semg0, semg1, semw0, semw1):
    core = lax.axis_index("c")
    sub = lax.axis_index("s")
    wid = sub * NCORE + core
    base = wid * GPW_G * CHUNK
    srcv = (srcv0, srcv1)
    dstv = (dstv0, dstv1)
    r3 = (r30, r31)
    r4 = (r40, r41)
    semi = (semi0, semi1)
    semg = (semg0, semg1)
    semw = (semw0, semw1)

    def body(it, carry):
        offs = [pl.multiple_of(base + (it * 2 + b) * SUPE, SUPE)
                for b in range(2)]
        icps = []
        for b in range(2):
            icps.append(pltpu.async_copy(
                src_h.at[pl.ds(offs[b], SUPE)], srcv[b], semi[b]))
            icps.append(pltpu.async_copy(
                dst_h.at[pl.ds(offs[b], SUPE)], dstv[b], semi[b]))
        gcps = [[], []]
        for b in range(2):
            @pl.when(it > 0)
            def _(b=b):
                pltpu.make_async_copy(
                    r3[b], g_h.at[pl.ds(offs[b], SUPE)], semw[b]).wait()

            icps[2 * b].wait()
            icps[2 * b + 1].wait()
            for k in range(SUPE // CHUNK):
                sl = pl.ds(k * CHUNK, CHUNK)
                gcps[b].append(pltpu.async_copy(
                    x3_h.at[srcv[b].at[sl]], r3[b].at[sl, :], semg[b]))
                gcps[b].append(pltpu.async_copy(
                    x4_h.at[dstv[b].at[sl]], r4[b].at[sl, :], semg[b]))
        for b in range(2):
            for cp in gcps[b]:
                cp.wait()

            def add_row(i, c, b=b):
                for kk in range(UNITS // 16):
                    sl = pl.ds(kk * 16, 16)
                    r3[b][i, sl] = r3[b][i, sl] + r4[b][i, sl]
                return c

            lax.fori_loop(0, SUPE, add_row, 0, unroll=4)
            pltpu.async_copy(r3[b], g_h.at[pl.ds(offs[b], SUPE)], semw[b])
        return carry

    lax.fori_loop(0, NOUT_G, body, 0)
    for b in range(2):
        pltpu.make_async_copy(r3[b], g_h.at[pl.ds(base, SUPE)], semw[b]).wait()


@functools.cache
def _sc_agg_fn():
    return functools.partial(
        pl.kernel,
        mesh=_mesh(),
        compiler_params=pltpu.CompilerParams(use_tc_tiling_on_sc=False),
        out_type=jax.ShapeDtypeStruct((N_NODES, UNITS), f32),
        scratch_types=[
            pltpu.VMEM((SUPE_A,), jnp.int32),
            pltpu.VMEM((SUPE_A,), jnp.int32),
            pltpu.VMEM((SUPE_A,), jnp.int32),
            pltpu.VMEM((SUPE_A,), jnp.int32),
            pltpu.VMEM((SUPE_A,), jnp.int32),
            pltpu.VMEM((SUPE_A,), jnp.int32),
            pltpu.VMEM((SUPE_A, UNITS), f32),
            pltpu.VMEM((SUPE_A, UNITS), f32),
            pltpu.VMEM((SUPE_A, UNITS), f32),
            pltpu.VMEM((SUPE_A, UNITS), f32),
            pltpu.VMEM_SHARED((ROWS_PAD, UNITS), f32),
            pltpu.SemaphoreType.DMA,
            pltpu.SemaphoreType.DMA,
            pltpu.SemaphoreType.DMA,
            pltpu.SemaphoreType.DMA,
        ],
    )(_sc_agg_body)


def _sc_agg_body(src_h, dst_h, x2_h, s_h, zeros_h, out_h,
                 srcv0, srcv1, dstv0, dstv1, lidx0, lidx1,
                 r20, r21, sv0, sv1, aggS,
                 semi0, semi1, semg0, semg1):
    core = lax.axis_index("c")
    sub = lax.axis_index("s")
    base = sub * GPW_A * CHUNK
    srcv = (srcv0, srcv1)
    dstv = (dstv0, dstv1)
    lidx = (lidx0, lidx1)
    r2 = (r20, r21)
    sv = (sv0, sv1)
    semi = (semi0, semi1)
    semg = (semg0, semg1)
    pltpu.sync_copy(zeros_h, aggS.at[pl.ds(sub * RPS, RPS)])
    plsc.subcore_barrier()

    def body(it, carry):
        offs = [pl.multiple_of(base + (it * 2 + b) * SUPE_A, SUPE_A)
                for b in range(2)]
        icps = []
        for b in range(2):
            icps.append(pltpu.async_copy(
                src_h.at[pl.ds(offs[b], SUPE_A)], srcv[b], semi[b]))
            icps.append(pltpu.async_copy(
                dst_h.at[pl.ds(offs[b], SUPE_A)], dstv[b], semi[b]))
            icps.append(pltpu.async_copy(
                s_h.at[pl.ds(offs[b], SUPE_A)], sv[b], semi[b]))
        gcps = [[], []]
        for b in range(2):
            icps[3 * b].wait()
            icps[3 * b + 1].wait()
            icps[3 * b + 2].wait()
            for k in range(SUPE_A // CHUNK):
                sl = pl.ds(k * CHUNK, CHUNK)
                gcps[b].append(pltpu.async_copy(
                    x2_h.at[dstv[b].at[sl]], r2[b].at[sl, :], semg[b]))
            for kk in range(SUPE_A // 16):
                sl = pl.ds(kk * 16, 16)
                v = srcv[b][sl] - core * HALF
                ok = (v >= 0) & (v < HALF)
                lidx[b][sl] = jnp.where(ok, v, HALF)
        for b in range(2):
            for cp in gcps[b]:
                cp.wait()

            def mul_row(i, c, b=b):
                for kk in range(UNITS // 16):
                    sl = pl.ds(kk * 16, 16)
                    r2[b][i, sl] = r2[b][i, sl] * sv[b][i, sl]
                return c

            lax.fori_loop(0, SUPE_A, mul_row, 0, unroll=4)
            pltpu.sync_copy(r2[b], aggS.at[lidx[b]], add=True)
        return carry

    lax.fori_loop(0, NOUT_A, body, 0)
    plsc.subcore_barrier()

    @pl.when(sub < NSUB - 1)
    def _():
        pltpu.sync_copy(
            aggS.at[pl.ds(sub * RPS, RPS)],
            out_h.at[pl.ds(core * HALF + sub * RPS, RPS)],
        )

    @pl.when(sub == NSUB - 1)
    def _():
        pltpu.sync_copy(
            aggS.at[pl.ds((NSUB - 1) * RPS, RPS_LAST)],
            out_h.at[pl.ds(core * HALF + (NSUB - 1) * RPS, RPS_LAST)],
        )


EBLK = 10000
EGRID = N_EDGES // EBLK


def _ninit_body(nodes, v0W, v0b, vRW, vRb, posW, R, cosb, sinb, maskb, alpha,
                xo):
    i = pl.program_id(0)
    nb = nodes[...]
    xb = jnp.dot(nb, v0W[...], preferred_element_type=f32) + v0b[...]

    @pl.when(i < MP // 5000)
    def _():
        place = jnp.dot(nb, vRW[...], preferred_element_type=f32) + vRb[...]
        p2 = jnp.dot(place, R[...], preferred_element_type=f32)
        pe = place * cosb[...] + p2 * sinb[...]
        pe = jnp.dot(pe, posW[...], preferred_element_type=f32) * alpha[0, 0]
        xo[...] = _silu(xb + pe * maskb[...])

    @pl.when(i >= MP // 5000)
    def _():
        xo[...] = _silu(xb)


def _node_init(nodes, v0W, v0b, vRW, vRb, posW, R, cosb, sinb, maskb, alpha):
    blk = 5000
    pspec = pl.BlockSpec((blk, UNITS), lambda i: (jnp.minimum(i, 1), 0))
    return pl.pallas_call(
        _ninit_body,
        grid=(N_NODES // blk,),
        in_specs=[
            pl.BlockSpec((blk, 2), lambda i: (i, 0)),
            pl.BlockSpec((2, UNITS), lambda i: (0, 0)),
            pl.BlockSpec((1, UNITS), lambda i: (0, 0)),
            pl.BlockSpec((2, UNITS), lambda i: (0, 0)),
            pl.BlockSpec((1, UNITS), lambda i: (0, 0)),
            pl.BlockSpec((UNITS, UNITS), lambda i: (0, 0)),
            pl.BlockSpec((UNITS, UNITS), lambda i: (0, 0)),
            pspec,
            pspec,
            pspec,
            pl.BlockSpec((1, 1), lambda i: (0, 0)),
        ],
        out_specs=pl.BlockSpec((blk, UNITS), lambda i: (i, 0)),
        out_shape=jax.ShapeDtypeStruct((N_NODES, UNITS), f32),
    )(nodes, v0W, v0b, vRW, vRb, posW, R, cosb, sinb, maskb, alpha)


def _einit_body(ea, e0W, e0b, w_ref, s_ref):
    z = ea[...] * e0W[...] + e0b[...]
    wv = _silu(z)
    w_ref[...] = wv
    s_ref[...] = _sig(wv)


def _edge_init(ea, e0W, e0b):
    return pl.pallas_call(
        _einit_body,
        grid=(EGRID,),
        in_specs=[
            pl.BlockSpec((EBLK, 1), lambda i: (i, 0)),
            pl.BlockSpec((1, UNITS), lambda i: (0, 0)),
            pl.BlockSpec((1, UNITS), lambda i: (0, 0)),
        ],
        out_specs=[
            pl.BlockSpec((EBLK, UNITS), lambda i: (i, 0)),
            pl.BlockSpec((EBLK, UNITS), lambda i: (i, 0)),
        ],
        out_shape=[jax.ShapeDtypeStruct((N_EDGES, UNITS), f32),
                   jax.ShapeDtypeStruct((EPAD, UNITS), f32)],
    )(ea, e0W, e0b)


NBLK = 5000
NGRID = N_NODES // NBLK


def _nmm_body(x, Wc, bc, o1, o2, o3, o4):
    acc = jnp.dot(x[...], Wc[...], preferred_element_type=f32) + bc[...]
    o1[...] = acc[:, 0 * UNITS:1 * UNITS]
    o2[...] = acc[:, 1 * UNITS:2 * UNITS]
    o3[...] = acc[:, 2 * UNITS:3 * UNITS]
    o4[...] = acc[:, 3 * UNITS:4 * UNITS]


def _node_mm4(x, Wc, bc):
    nb = jax.ShapeDtypeStruct((NPAD, UNITS), f32)
    nspec = pl.BlockSpec((NBLK, UNITS), lambda i: (i, 0))
    return pl.pallas_call(
        _nmm_body,
        grid=(NGRID,),
        in_specs=[
            nspec,
            pl.BlockSpec((UNITS, 4 * UNITS), lambda i: (0, 0)),
            pl.BlockSpec((1, 4 * UNITS), lambda i: (0, 0)),
        ],
        out_specs=[nspec, nspec, nspec, nspec],
        out_shape=[nb, nb, nb, nb],
    )(x, Wc, bc)


def _emm_body(w0, g, Wr, br, t_ref, sums_ref):
    pid = pl.program_id(0)
    t = (jnp.dot(w0[...], Wr[...], preferred_element_type=f32) + br[...]
         + g[...])
    t_ref[...] = t

    @pl.when(pid == 0)
    def _():
        sums_ref[...] = jnp.zeros_like(sums_ref)

    sums_ref[0:1, :] += jnp.sum(t, axis=0, keepdims=True)
    sums_ref[1:2, :] += jnp.sum(t * t, axis=0, keepdims=True)


def _edge_mm_stats(w0, g, Wr, br):
    return pl.pallas_call(
        _emm_body,
        grid=(EGRID,),
        in_specs=[
            pl.BlockSpec((EBLK, UNITS), lambda i: (i, 0)),
            pl.BlockSpec((EBLK, UNITS), lambda i: (i, 0)),
            pl.BlockSpec((UNITS, UNITS), lambda i: (0, 0)),
            pl.BlockSpec((1, UNITS), lambda i: (0, 0)),
        ],
        out_specs=[
            pl.BlockSpec((EBLK, UNITS), lambda i: (i, 0)),
            pl.BlockSpec((2, UNITS), lambda i: (0, 0)),
        ],
        out_shape=[
            jax.ShapeDtypeStruct((N_EDGES, UNITS), f32),
            jax.ShapeDtypeStruct((2, UNITS), f32),
        ],
    )(w0, g, Wr, br)


def _nstat_body(x1, agg, cnt, t_ref, sums_ref):
    pid = pl.program_id(0)
    cn = cnt[...]
    a = jnp.where(cn > 0, agg[...] / jnp.maximum(cn, 1.0), 0.0)
    t = x1[...] + a
    t_ref[...] = t

    @pl.when(pid == 0)
    def _():
        sums_ref[...] = jnp.zeros_like(sums_ref)

    sums_ref[0:1, :] += jnp.sum(t, axis=0, keepdims=True)
    sums_ref[1:2, :] += jnp.sum(t * t, axis=0, keepdims=True)


def _napply_body(t, x0, sums, gm, bt, xo):
    inv_n = 1.0 / N_NODES
    mu = sums[0:1, :] * inv_n
    var = sums[1:2, :] * inv_n - mu * mu
    y = (t[...] - mu) * lax.rsqrt(var + 1e-5) * gm[...] + bt[...]
    xo[...] = x0[...] + _silu(y)


def _node_update(x1, agg, cnt, x0, gm, bt):
    nb = jax.ShapeDtypeStruct((N_NODES, UNITS), f32)
    nspec = pl.BlockSpec((NBLK, UNITS), lambda i: (i, 0))
    cspec = pl.BlockSpec((1, UNITS), lambda i: (0, 0))
    t, sums = pl.pallas_call(
        _nstat_body,
        grid=(NGRID,),
        in_specs=[nspec, nspec, nspec],
        out_specs=[nspec, pl.BlockSpec((2, UNITS), lambda i: (0, 0))],
        out_shape=[nb, jax.ShapeDtypeStruct((2, UNITS), f32)],
    )(x1, agg, cnt)
    return pl.pallas_call(
        _napply_body,
        grid=(NGRID,),
        in_specs=[nspec, nspec,
                  pl.BlockSpec((2, UNITS), lambda i: (0, 0)), cspec, cspec],
        out_specs=nspec,
        out_shape=nb,
    )(t, x0, sums, gm, bt)


def _eapply_body(t, w0, sums, gm, bt, w_ref, s_ref):
    inv_n = 1.0 / N_EDGES
    mu = sums[0:1, :] * inv_n
    var = sums[1:2, :] * inv_n - mu * mu
    y = (t[...] - mu) * lax.rsqrt(var + 1e-5) * gm[...] + bt[...]
    wn = w0[...] + _silu(y)
    w_ref[...] = wn
    s_ref[...] = _sig(wn)


def _edge_apply(t, w0, sums, gm, bt):
    eb = jax.ShapeDtypeStruct((N_EDGES, UNITS), f32)
    sb = jax.ShapeDtypeStruct((EPAD, UNITS), f32)
    return pl.pallas_call(
        _eapply_body,
        grid=(EGRID,),
        in_specs=[
            pl.BlockSpec((EBLK, UNITS), lambda i: (i, 0)),
            pl.BlockSpec((EBLK, UNITS), lambda i: (i, 0)),
            pl.BlockSpec((2, UNITS), lambda i: (0, 0)),
            pl.BlockSpec((1, UNITS), lambda i: (0, 0)),
            pl.BlockSpec((1, UNITS), lambda i: (0, 0)),
        ],
        out_specs=[
            pl.BlockSpec((EBLK, UNITS), lambda i: (i, 0)),
            pl.BlockSpec((EBLK, UNITS), lambda i: (i, 0)),
        ],
        out_shape=[eb, sb],
    )(t, w0, sums, gm, bt)


def kernel(nodes, edge_index, edge_attr, M, alpha, pos_W, v0_W, v0_b, vR_W,
           vR_b, e0_W, e0_b, V1_W, V1_b, V2_W, V2_b, V3_W, V3_b, V4_W, V4_b,
           Ew_W, Ew_b, VBN_g, VBN_b, EBN_g, EBN_b):
    src = edge_index[0]
    dst = edge_index[1]
    npad = EPAD - N_EDGES
    srcp = jnp.concatenate(
        [src, jnp.full((npad,), N_NODES, dtype=jnp.int32)])
    dstp = jnp.concatenate([dst, jnp.zeros((npad,), dtype=jnp.int32)])

    ids = jnp.arange(UNITS // 2, dtype=f32)
    theta = jnp.power(1000.0, -2.0 * ids / UNITS)
    emb = jnp.arange(MP, dtype=f32)[:, None] * theta
    cosb = jnp.repeat(jnp.sin(emb), 2, axis=-1)
    sinb = jnp.repeat(jnp.cos(emb), 2, axis=-1)
    maskb = jnp.broadcast_to(
        (jnp.arange(MP) < M)[:, None], (MP, UNITS)).astype(f32)
    k2 = jnp.arange(UNITS // 2)
    R = (jnp.zeros((UNITS, UNITS), f32)
         .at[2 * k2 + 1, 2 * k2].set(-1.0)
         .at[2 * k2, 2 * k2 + 1].set(1.0))

    ones_h = jnp.ones((SUPE_C, UNITS), f32)
    zeros_h = jnp.zeros((RPS, UNITS), f32)

    x = _node_init(nodes, v0_W, v0_b.reshape(1, -1), vR_W,
                   vR_b.reshape(1, -1), pos_W, R, cosb, sinb, maskb,
                   alpha.reshape(1, 1))
    w, s = _edge_init(edge_attr, e0_W, e0_b.reshape(1, -1))
    cntf = _sc_cnt_fn()(srcp, ones_h, zeros_h)

    for i in range(DEPTH):
        Wc = jnp.concatenate([V1_W[i], V2_W[i], V3_W[i], V4_W[i]], axis=1)
        bc = jnp.concatenate([V1_b[i], V2_b[i], V3_b[i], V4_b[i]]).reshape(1, -1)
        x1, x2, x3, x4 = _node_mm4(x, Wc, bc)
        g = _sc_gsum_fn()(srcp, dstp, x3, x4)
        aggr = _sc_agg_fn()(srcp, dstp, x2, s, zeros_h)
        t, sums = _edge_mm_stats(w, g, Ew_W[i], Ew_b[i].reshape(1, -1))
        x = _node_update(x1, aggr, cntf, x, VBN_g[i].reshape(1, -1),
                         VBN_b[i].reshape(1, -1))
        w, s = _edge_apply(t, w, sums, EBN_g[i].reshape(1, -1),
                           EBN_b[i].reshape(1, -1))
    return (x, w)


# speedup vs baseline: 1.8065x; 1.0375x over previous
import functools

import jax
import jax.numpy as jnp
from jax import lax
from jax.experimental import pallas as pl
from jax.experimental.pallas import tpu as pltpu
from jax.experimental.pallas import tpu_sc as plsc

N_NODES = 50000
N_EDGES = 800000
UNITS = 48
DEPTH = 12
MP = 10000

NCORE = 2
NSUB = 16
CHUNK = 128
NCH = N_EDGES // CHUNK
HALF = N_NODES // 2
ROWS_PAD = 25008
RPS = ROWS_PAD // NSUB
RPS_LAST = HALF - RPS * (NSUB - 1)

EPAD = 819200
NPAD = 50008
SUPE = 512
GPW_G = EPAD // CHUNK // (NCORE * NSUB)
NOUT_G = GPW_G * CHUNK // SUPE // 2
GPW_A = EPAD // CHUNK // NSUB
SUPE_A = 256
NOUT_A = GPW_A * CHUNK // SUPE_A // 2

f32 = jnp.float32
bf16 = jnp.bfloat16
UB = 64


@functools.cache
def _mesh():
    return plsc.VectorSubcoreMesh(core_axis_name="c", subcore_axis_name="s")


def _sig(z):
    return 1.0 / (1.0 + jnp.exp(-z))


def _silu(z):
    return z * _sig(z)


SUPE_C = 512
NOUT_C = GPW_A * CHUNK // SUPE_C // 2


@functools.cache
def _sc_cnt_fn():
    return functools.partial(
        pl.kernel,
        mesh=_mesh(),
        compiler_params=pltpu.CompilerParams(use_tc_tiling_on_sc=False),
        out_type=jax.ShapeDtypeStruct((N_NODES, UNITS), f32),
        scratch_types=[
            pltpu.VMEM((SUPE_C,), jnp.int32),
            pltpu.VMEM((SUPE_C,), jnp.int32),
            pltpu.VMEM((SUPE_C,), jnp.int32),
            pltpu.VMEM((SUPE_C,), jnp.int32),
            pltpu.VMEM((SUPE_C, UNITS), f32),
            pltpu.VMEM_SHARED((ROWS_PAD, UNITS), f32),
            pltpu.SemaphoreType.DMA,
            pltpu.SemaphoreType.DMA,
        ],
    )(_sc_cnt_body)


def _sc_cnt_body(src_h, ones_h, zeros_h, out_h,
                 srcv0, srcv1, lidx0, lidx1, ones_v, aggS, semi0, semi1):
    core = lax.axis_index("c")
    sub = lax.axis_index("s")
    base = sub * GPW_A * CHUNK
    srcv = (srcv0, srcv1)
    lidx = (lidx0, lidx1)
    semi = (semi0, semi1)
    pltpu.sync_copy(zeros_h, aggS.at[pl.ds(sub * RPS, RPS)])
    pltpu.sync_copy(ones_h, ones_v)
    plsc.subcore_barrier()

    def body(it, carry):
        offs = [pl.multiple_of(base + (it * 2 + b) * SUPE_C, SUPE_C)
                for b in range(2)]
        icps = [pltpu.async_copy(
            src_h.at[pl.ds(offs[b], SUPE_C)], srcv[b], semi[b])
            for b in range(2)]
        for b in range(2):
            icps[b].wait()
            for kk in range(SUPE_C // 16):
                sl = pl.ds(kk * 16, 16)
                v = srcv[b][sl] - core * HALF
                ok = (v >= 0) & (v < HALF)
                lidx[b][sl] = jnp.where(ok, v, HALF)
            pltpu.sync_copy(ones_v, aggS.at[lidx[b]], add=True)
        return carry

    lax.fori_loop(0, NOUT_C, body, 0)
    plsc.subcore_barrier()

    @pl.when(sub < NSUB - 1)
    def _():
        pltpu.sync_copy(
            aggS.at[pl.ds(sub * RPS, RPS)],
            out_h.at[pl.ds(core * HALF + sub * RPS, RPS)],
        )

    @pl.when(sub == NSUB - 1)
    def _():
        pltpu.sync_copy(
            aggS.at[pl.ds((NSUB - 1) * RPS, RPS_LAST)],
            out_h.at[pl.ds(core * HALF + (NSUB - 1) * RPS, RPS_LAST)],
        )


@functools.cache
def _sc_gsum_fn():
    return functools.partial(
        pl.kernel,
        mesh=_mesh(),
        compiler_params=pltpu.CompilerParams(use_tc_tiling_on_sc=False),
        out_type=jax.ShapeDtypeStruct((EPAD, UB), bf16),
        scratch_types=[
            pltpu.VMEM((SUPE,), jnp.int32),
            pltpu.VMEM((SUPE,), jnp.int32),
            pltpu.VMEM((SUPE,), jnp.int32),
            pltpu.VMEM((SUPE,), jnp.int32),
            pltpu.VMEM((SUPE, UB), bf16),
            pltpu.VMEM((SUPE, UB), bf16),
            pltpu.VMEM((SUPE, UB), bf16),
            pltpu.VMEM((SUPE, UB), bf16),
            pltpu.SemaphoreType.DMA,
            pltpu.SemaphoreType.DMA,
            pltpu.SemaphoreType.DMA,
            pltpu.SemaphoreType.DMA,
            pltpu.SemaphoreType.DMA,
            pltpu.SemaphoreType.DMA,
        ],
    )(_sc_gsum_body)


def _sc_gsum_body(src_h, dst_h, x3_h, x4_h, g_h,
                  srcv0, srcv1, dstv0, dstv1, r30, r31, r40, r41,
                  semi0, semi1, semg0, semg1, semw0, semw1):
    core = lax.axis_index("c")
    sub = lax.axis_index("s")
    wid = sub * NCORE + core
    base = wid * GPW_G * CHUNK
    srcv = (srcv0, srcv1)
    dstv = (dstv0, dstv1)
    r3 = (r30, r31)
    r4 = (r40, r41)
    semi = (semi0, semi1)
    semg = (semg0, semg1)
    semw = (semw0, semw1)

    def body(it, carry):
        offs = [pl.multiple_of(base + (it * 2 + b) * SUPE, SUPE)
                for b in range(2)]
        icps = []
        for b in range(2):
            icps.append(pltpu.async_copy(
                src_h.at[pl.ds(offs[b], SUPE)], srcv[b], semi[b]))
            icps.append(pltpu.async_copy(
                dst_h.at[pl.ds(offs[b], SUPE)], dstv[b], semi[b]))
        gcps = [[], []]
        for b in range(2):
            @pl.when(it > 0)
            def _(b=b):
                pltpu.make_async_copy(
                    r3[b], g_h.at[pl.ds(offs[b], SUPE)], semw[b]).wait()

            icps[2 * b].wait()
            icps[2 * b + 1].wait()
            for k in range(SUPE // CHUNK):
                sl = pl.ds(k * CHUNK, CHUNK)
                gcps[b].append(pltpu.async_copy(
                    x3_h.at[srcv[b].at[sl]], r3[b].at[sl, :], semg[b]))
                gcps[b].append(pltpu.async_copy(
                    x4_h.at[dstv[b].at[sl]], r4[b].at[sl, :], semg[b]))
        for b in range(2):
            for cp in gcps[b]:
                cp.wait()

            def add_row(i, c, b=b):
                for kk in range(UB // 32):
                    sl = pl.ds(kk * 32, 32)
                    r3[b][i, sl] = r3[b][i, sl] + r4[b][i, sl]
                return c

            lax.fori_loop(0, SUPE, add_row, 0, unroll=4)
            pltpu.async_copy(r3[b], g_h.at[pl.ds(offs[b], SUPE)], semw[b])
        return carry

    lax.fori_loop(0, NOUT_G, body, 0)
    for b in range(2):
        pltpu.make_async_copy(r3[b], g_h.at[pl.ds(base, SUPE)], semw[b]).wait()


@functools.cache
def _sc_agg_fn():
    return functools.partial(
        pl.kernel,
        mesh=_mesh(),
        compiler_params=pltpu.CompilerParams(use_tc_tiling_on_sc=False),
        out_type=jax.ShapeDtypeStruct((N_NODES, UNITS), f32),
        scratch_types=[
            pltpu.VMEM((SUPE_A,), jnp.int32),
            pltpu.VMEM((SUPE_A,), jnp.int32),
            pltpu.VMEM((SUPE_A,), jnp.int32),
            pltpu.VMEM((SUPE_A,), jnp.int32),
            pltpu.VMEM((SUPE_A,), jnp.int32),
            pltpu.VMEM((SUPE_A,), jnp.int32),
            pltpu.VMEM((SUPE_A, UNITS), f32),
            pltpu.VMEM((SUPE_A, UNITS), f32),
            pltpu.VMEM((SUPE_A, UNITS), f32),
            pltpu.VMEM((SUPE_A, UNITS), f32),
            pltpu.VMEM_SHARED((ROWS_PAD, UNITS), f32),
            pltpu.SemaphoreType.DMA,
            pltpu.SemaphoreType.DMA,
            pltpu.SemaphoreType.DMA,
            pltpu.SemaphoreType.DMA,
        ],
    )(_sc_agg_body)


def _sc_agg_body(src_h, dst_h, x2_h, s_h, zeros_h, out_h,
                 srcv0, srcv1, dstv0, dstv1, lidx0, lidx1,
                 r20, r21, sv0, sv1, aggS,
                 semi0, semi1, semg0, semg1):
    core = lax.axis_index("c")
    sub = lax.axis_index("s")
    base = sub * GPW_A * CHUNK
    srcv = (srcv0, srcv1)
    dstv = (dstv0, dstv1)
    lidx = (lidx0, lidx1)
    r2 = (r20, r21)
    sv = (sv0, sv1)
    semi = (semi0, semi1)
    semg = (semg0, semg1)
    pltpu.sync_copy(zeros_h, aggS.at[pl.ds(sub * RPS, RPS)])
    plsc.subcore_barrier()

    def body(it, carry):
        offs = [pl.multiple_of(base + (it * 2 + b) * SUPE_A, SUPE_A)
                for b in range(2)]
        icps = []
        for b in range(2):
            icps.append(pltpu.async_copy(
                src_h.at[pl.ds(offs[b], SUPE_A)], srcv[b], semi[b]))
            icps.append(pltpu.async_copy(
                dst_h.at[pl.ds(offs[b], SUPE_A)], dstv[b], semi[b]))
            icps.append(pltpu.async_copy(
                s_h.at[pl.ds(offs[b], SUPE_A)], sv[b], semi[b]))
        gcps = [[], []]
        for b in range(2):
            icps[3 * b].wait()
            icps[3 * b + 1].wait()
            icps[3 * b + 2].wait()
            for k in range(SUPE_A // CHUNK):
                sl = pl.ds(k * CHUNK, CHUNK)
                gcps[b].append(pltpu.async_copy(
                    x2_h.at[dstv[b].at[sl]], r2[b].at[sl, :], semg[b]))
            for kk in range(SUPE_A // 16):
                sl = pl.ds(kk * 16, 16)
                v = srcv[b][sl] - core * HALF
                ok = (v >= 0) & (v < HALF)
                lidx[b][sl] = jnp.where(ok, v, HALF)
        for b in range(2):
            for cp in gcps[b]:
                cp.wait()

            def mul_row(i, c, b=b):
                for kk in range(UNITS // 16):
                    sl = pl.ds(kk * 16, 16)
                    r2[b][i, sl] = r2[b][i, sl] * sv[b][i, sl]
                return c

            lax.fori_loop(0, SUPE_A, mul_row, 0, unroll=4)
            pltpu.sync_copy(r2[b], aggS.at[lidx[b]], add=True)
        return carry

    lax.fori_loop(0, NOUT_A, body, 0)
    plsc.subcore_barrier()

    @pl.when(sub < NSUB - 1)
    def _():
        pltpu.sync_copy(
            aggS.at[pl.ds(sub * RPS, RPS)],
            out_h.at[pl.ds(core * HALF + sub * RPS, RPS)],
        )

    @pl.when(sub == NSUB - 1)
    def _():
        pltpu.sync_copy(
            aggS.at[pl.ds((NSUB - 1) * RPS, RPS_LAST)],
            out_h.at[pl.ds(core * HALF + (NSUB - 1) * RPS, RPS_LAST)],
        )


EBLK = 10000
EGRID = N_EDGES // EBLK


def _ninit_body(nodes, v0W, v0b, vRW, vRb, posW, R, cosb, sinb, maskb, alpha,
                xo):
    i = pl.program_id(0)
    nb = nodes[...]
    xb = jnp.dot(nb, v0W[...], preferred_element_type=f32) + v0b[...]

    @pl.when(i < MP // 5000)
    def _():
        place = jnp.dot(nb, vRW[...], preferred_element_type=f32) + vRb[...]
        p2 = jnp.dot(place, R[...], preferred_element_type=f32)
        pe = place * cosb[...] + p2 * sinb[...]
        pe = jnp.dot(pe, posW[...], preferred_element_type=f32) * alpha[0, 0]
        xo[...] = _silu(xb + pe * maskb[...])

    @pl.when(i >= MP // 5000)
    def _():
        xo[...] = _silu(xb)


def _node_init(nodes, v0W, v0b, vRW, vRb, posW, R, cosb, sinb, maskb, alpha):
    blk = 5000
    pspec = pl.BlockSpec((blk, UNITS), lambda i: (jnp.minimum(i, 1), 0))
    return pl.pallas_call(
        _ninit_body,
        grid=(N_NODES // blk,),
        in_specs=[
            pl.BlockSpec((blk, 2), lambda i: (i, 0)),
            pl.BlockSpec((2, UNITS), lambda i: (0, 0)),
            pl.BlockSpec((1, UNITS), lambda i: (0, 0)),
            pl.BlockSpec((2, UNITS), lambda i: (0, 0)),
            pl.BlockSpec((1, UNITS), lambda i: (0, 0)),
            pl.BlockSpec((UNITS, UNITS), lambda i: (0, 0)),
            pl.BlockSpec((UNITS, UNITS), lambda i: (0, 0)),
            pspec,
            pspec,
            pspec,
            pl.BlockSpec((1, 1), lambda i: (0, 0)),
        ],
        out_specs=pl.BlockSpec((blk, UNITS), lambda i: (i, 0)),
        out_shape=jax.ShapeDtypeStruct((N_NODES, UNITS), f32),
    )(nodes, v0W, v0b, vRW, vRb, posW, R, cosb, sinb, maskb, alpha)


def _einit_body(ea, e0W, e0b, w_ref, s_ref):
    z = ea[...] * e0W[...] + e0b[...]
    wv = _silu(z)
    w_ref[...] = wv
    s_ref[...] = _sig(wv)


def _edge_init(ea, e0W, e0b):
    return pl.pallas_call(
        _einit_body,
        grid=(EGRID,),
        in_specs=[
            pl.BlockSpec((EBLK, 1), lambda i: (i, 0)),
            pl.BlockSpec((1, UNITS), lambda i: (0, 0)),
            pl.BlockSpec((1, UNITS), lambda i: (0, 0)),
        ],
        out_specs=[
            pl.BlockSpec((EBLK, UNITS), lambda i: (i, 0)),
            pl.BlockSpec((EBLK, UNITS), lambda i: (i, 0)),
        ],
        out_shape=[jax.ShapeDtypeStruct((N_EDGES, UNITS), f32),
                   jax.ShapeDtypeStruct((EPAD, UNITS), f32)],
    )(ea, e0W, e0b)


NBLK = 5000
NGRID = N_NODES // NBLK


NBLK_M = 10000


def _nmm_body(x, Wc, bc, o1, o2, o3, o4):
    acc = jnp.dot(x[...], Wc[...], preferred_element_type=f32) + bc[...]
    o1[...] = acc[:, 0 * UNITS:1 * UNITS]
    o2[...] = acc[:, 1 * UNITS:2 * UNITS]
    zpad = jnp.zeros((acc.shape[0], UB - UNITS), dtype=bf16)
    o3[...] = jnp.concatenate(
        [acc[:, 2 * UNITS:3 * UNITS].astype(bf16), zpad], axis=1)
    o4[...] = jnp.concatenate(
        [acc[:, 3 * UNITS:4 * UNITS].astype(bf16), zpad], axis=1)


def _node_mm4(x, Wc, bc):
    nb = jax.ShapeDtypeStruct((NPAD, UNITS), f32)
    hb = jax.ShapeDtypeStruct((NPAD, UB), bf16)
    nspec = pl.BlockSpec((NBLK_M, UNITS), lambda i: (i, 0))
    hspec = pl.BlockSpec((NBLK_M, UB), lambda i: (i, 0))
    return pl.pallas_call(
        _nmm_body,
        grid=(N_NODES // NBLK_M,),
        in_specs=[
            nspec,
            pl.BlockSpec((UNITS, 4 * UNITS), lambda i: (0, 0)),
            pl.BlockSpec((1, 4 * UNITS), lambda i: (0, 0)),
        ],
        out_specs=[nspec, nspec, hspec, hspec],
        out_shape=[nb, nb, hb, hb],
    )(x, Wc, bc)


def _emm_body(w0, g, Wr, br, t_ref, sums_ref):
    pid = pl.program_id(0)
    t = (jnp.dot(w0[...], Wr[...], preferred_element_type=f32) + br[...]
         + g[:, 0:UNITS].astype(f32))
    t_ref[...] = t

    @pl.when(pid == 0)
    def _():
        sums_ref[...] = jnp.zeros_like(sums_ref)

    sums_ref[0:1, :] += jnp.sum(t, axis=0, keepdims=True)
    sums_ref[1:2, :] += jnp.sum(t * t, axis=0, keepdims=True)


def _edge_mm_stats(w0, g, Wr, br):
    return pl.pallas_call(
        _emm_body,
        grid=(EGRID,),
        in_specs=[
            pl.BlockSpec((EBLK, UNITS), lambda i: (i, 0)),
            pl.BlockSpec((EBLK, UB), lambda i: (i, 0)),
            pl.BlockSpec((UNITS, UNITS), lambda i: (0, 0)),
            pl.BlockSpec((1, UNITS), lambda i: (0, 0)),
        ],
        out_specs=[
            pl.BlockSpec((EBLK, UNITS), lambda i: (i, 0)),
            pl.BlockSpec((2, UNITS), lambda i: (0, 0)),
        ],
        out_shape=[
            jax.ShapeDtypeStruct((N_EDGES, UNITS), f32),
            jax.ShapeDtypeStruct((2, UNITS), f32),
        ],
    )(w0, g, Wr, br)


def _nstat_body(x1, agg, cnt, t_ref, sums_ref):
    pid = pl.program_id(0)
    cn = cnt[...]
    a = jnp.where(cn > 0, agg[...] / jnp.maximum(cn, 1.0), 0.0)
    t = x1[...] + a
    t_ref[...] = t

    @pl.when(pid == 0)
    def _():
        sums_ref[...] = jnp.zeros_like(sums_ref)

    sums_ref[0:1, :] += jnp.sum(t, axis=0, keepdims=True)
    sums_ref[1:2, :] += jnp.sum(t * t, axis=0, keepdims=True)


def _napply_body(t, x0, sums, gm, bt, xo):
    inv_n = 1.0 / N_NODES
    mu = sums[0:1, :] * inv_n
    var = sums[1:2, :] * inv_n - mu * mu
    y = (t[...] - mu) * lax.rsqrt(var + 1e-5) * gm[...] + bt[...]
    xo[...] = x0[...] + _silu(y)


def _node_update(x1, agg, cnt, x0, gm, bt):
    nb = jax.ShapeDtypeStruct((N_NODES, UNITS), f32)
    nspec = pl.BlockSpec((NBLK, UNITS), lambda i: (i, 0))
    cspec = pl.BlockSpec((1, UNITS), lambda i: (0, 0))
    t, sums = pl.pallas_call(
        _nstat_body,
        grid=(NGRID,),
        in_specs=[nspec, nspec, nspec],
        out_specs=[nspec, pl.BlockSpec((2, UNITS), lambda i: (0, 0))],
        out_shape=[nb, jax.ShapeDtypeStruct((2, UNITS), f32)],
    )(x1, agg, cnt)
    return pl.pallas_call(
        _napply_body,
        grid=(NGRID,),
        in_specs=[nspec, nspec,
                  pl.BlockSpec((2, UNITS), lambda i: (0, 0)), cspec, cspec],
        out_specs=nspec,
        out_shape=nb,
    )(t, x0, sums, gm, bt)


def _eapply_body(t, w0, sums, gm, bt, w_ref, s_ref):
    inv_n = 1.0 / N_EDGES
    mu = sums[0:1, :] * inv_n
    var = sums[1:2, :] * inv_n - mu * mu
    y = (t[...] - mu) * lax.rsqrt(var + 1e-5) * gm[...] + bt[...]
    wn = w0[...] + _silu(y)
    w_ref[...] = wn
    s_ref[...] = _sig(wn)


def _edge_apply(t, w0, sums, gm, bt):
    eb = jax.ShapeDtypeStruct((N_EDGES, UNITS), f32)
    sb = jax.ShapeDtypeStruct((EPAD, UNITS), f32)
    return pl.pallas_call(
        _eapply_body,
        grid=(EGRID,),
        in_specs=[
            pl.BlockSpec((EBLK, UNITS), lambda i: (i, 0)),
            pl.BlockSpec((EBLK, UNITS), lambda i: (i, 0)),
            pl.BlockSpec((2, UNITS), lambda i: (0, 0)),
            pl.BlockSpec((1, UNITS), lambda i: (0, 0)),
            pl.BlockSpec((1, UNITS), lambda i: (0, 0)),
        ],
        out_specs=[
            pl.BlockSpec((EBLK, UNITS), lambda i: (i, 0)),
            pl.BlockSpec((EBLK, UNITS), lambda i: (i, 0)),
        ],
        out_shape=[eb, sb],
    )(t, w0, sums, gm, bt)


def kernel(nodes, edge_index, edge_attr, M, alpha, pos_W, v0_W, v0_b, vR_W,
           vR_b, e0_W, e0_b, V1_W, V1_b, V2_W, V2_b, V3_W, V3_b, V4_W, V4_b,
           Ew_W, Ew_b, VBN_g, VBN_b, EBN_g, EBN_b):
    src = edge_index[0]
    dst = edge_index[1]
    npad = EPAD - N_EDGES
    srcp = jnp.concatenate(
        [src, jnp.full((npad,), N_NODES, dtype=jnp.int32)])
    dstp = jnp.concatenate([dst, jnp.zeros((npad,), dtype=jnp.int32)])

    ids = jnp.arange(UNITS // 2, dtype=f32)
    theta = jnp.power(1000.0, -2.0 * ids / UNITS)
    emb = jnp.arange(MP, dtype=f32)[:, None] * theta
    cosb = jnp.repeat(jnp.sin(emb), 2, axis=-1)
    sinb = jnp.repeat(jnp.cos(emb), 2, axis=-1)
    maskb = jnp.broadcast_to(
        (jnp.arange(MP) < M)[:, None], (MP, UNITS)).astype(f32)
    k2 = jnp.arange(UNITS // 2)
    R = (jnp.zeros((UNITS, UNITS), f32)
         .at[2 * k2 + 1, 2 * k2].set(-1.0)
         .at[2 * k2, 2 * k2 + 1].set(1.0))

    ones_h = jnp.ones((SUPE_C, UNITS), f32)
    zeros_h = jnp.zeros((RPS, UNITS), f32)

    x = _node_init(nodes, v0_W, v0_b.reshape(1, -1), vR_W,
                   vR_b.reshape(1, -1), pos_W, R, cosb, sinb, maskb,
                   alpha.reshape(1, 1))
    w, s = _edge_init(edge_attr, e0_W, e0_b.reshape(1, -1))
    cntf = _sc_cnt_fn()(srcp, ones_h, zeros_h)

    for i in range(DEPTH):
        Wc = jnp.concatenate([V1_W[i], V2_W[i], V3_W[i], V4_W[i]], axis=1)
        bc = jnp.concatenate([V1_b[i], V2_b[i], V3_b[i], V4_b[i]]).reshape(1, -1)
        x1, x2, x3, x4 = _node_mm4(x, Wc, bc)
        g = _sc_gsum_fn()(srcp, dstp, x3, x4)
        aggr = _sc_agg_fn()(srcp, dstp, x2, s, zeros_h)
        t, sums = _edge_mm_stats(w, g, Ew_W[i], Ew_b[i].reshape(1, -1))
        x = _node_update(x1, aggr, cntf, x, VBN_g[i].reshape(1, -1),
                         VBN_b[i].reshape(1, -1))
        w, s = _edge_apply(t, w, sums, EBN_g[i].reshape(1, -1),
                           EBN_b[i].reshape(1, -1))
    return (x, w)


# speedup vs baseline: 1.8740x; 1.0374x over previous
import functools

import jax
import jax.numpy as jnp
from jax import lax
from jax.experimental import pallas as pl
from jax.experimental.pallas import tpu as pltpu
from jax.experimental.pallas import tpu_sc as plsc

N_NODES = 50000
N_EDGES = 800000
UNITS = 48
DEPTH = 12
MP = 10000

NCORE = 2
NSUB = 16
CHUNK = 128
NCH = N_EDGES // CHUNK
HALF = N_NODES // 2
ROWS_PAD = 25008
RPS = ROWS_PAD // NSUB
RPS_LAST = HALF - RPS * (NSUB - 1)

EPAD = 819200
NPAD = 50008
SUPE = 512
GPW_G = EPAD // CHUNK // (NCORE * NSUB)
NOUT_G = GPW_G * CHUNK // SUPE // 2
GPW_A = EPAD // CHUNK // NSUB
SUPE_A = 256
NOUT_A = GPW_A * CHUNK // SUPE_A // 2

f32 = jnp.float32
bf16 = jnp.bfloat16
UB = 64


@functools.cache
def _mesh():
    return plsc.VectorSubcoreMesh(core_axis_name="c", subcore_axis_name="s")


def _sig(z):
    return 1.0 / (1.0 + jnp.exp(-z))


def _silu(z):
    return z * _sig(z)


SUPE_C = 512
NOUT_C = GPW_A * CHUNK // SUPE_C // 2


@functools.cache
def _sc_cnt_fn():
    return functools.partial(
        pl.kernel,
        mesh=_mesh(),
        compiler_params=pltpu.CompilerParams(use_tc_tiling_on_sc=False),
        out_type=jax.ShapeDtypeStruct((N_NODES, UNITS), f32),
        scratch_types=[
            pltpu.VMEM((SUPE_C,), jnp.int32),
            pltpu.VMEM((SUPE_C,), jnp.int32),
            pltpu.VMEM((SUPE_C,), jnp.int32),
            pltpu.VMEM((SUPE_C,), jnp.int32),
            pltpu.VMEM((SUPE_C, UNITS), f32),
            pltpu.VMEM_SHARED((ROWS_PAD, UNITS), f32),
            pltpu.SemaphoreType.DMA,
            pltpu.SemaphoreType.DMA,
        ],
    )(_sc_cnt_body)


def _sc_cnt_body(src_h, ones_h, zeros_h, out_h,
                 srcv0, srcv1, lidx0, lidx1, ones_v, aggS, semi0, semi1):
    core = lax.axis_index("c")
    sub = lax.axis_index("s")
    base = sub * GPW_A * CHUNK
    srcv = (srcv0, srcv1)
    lidx = (lidx0, lidx1)
    semi = (semi0, semi1)
    pltpu.sync_copy(zeros_h, aggS.at[pl.ds(sub * RPS, RPS)])
    pltpu.sync_copy(ones_h, ones_v)
    plsc.subcore_barrier()

    def body(it, carry):
        offs = [pl.multiple_of(base + (it * 2 + b) * SUPE_C, SUPE_C)
                for b in range(2)]
        icps = [pltpu.async_copy(
            src_h.at[pl.ds(offs[b], SUPE_C)], srcv[b], semi[b])
            for b in range(2)]
        for b in range(2):
            icps[b].wait()
            for kk in range(SUPE_C // 16):
                sl = pl.ds(kk * 16, 16)
                v = srcv[b][sl] - core * HALF
                ok = (v >= 0) & (v < HALF)
                lidx[b][sl] = jnp.where(ok, v, HALF)
            pltpu.sync_copy(ones_v, aggS.at[lidx[b]], add=True)
        return carry

    lax.fori_loop(0, NOUT_C, body, 0)
    plsc.subcore_barrier()

    @pl.when(sub < NSUB - 1)
    def _():
        pltpu.sync_copy(
            aggS.at[pl.ds(sub * RPS, RPS)],
            out_h.at[pl.ds(core * HALF + sub * RPS, RPS)],
        )

    @pl.when(sub == NSUB - 1)
    def _():
        pltpu.sync_copy(
            aggS.at[pl.ds((NSUB - 1) * RPS, RPS_LAST)],
            out_h.at[pl.ds(core * HALF + (NSUB - 1) * RPS, RPS_LAST)],
        )


SUPE_M = 128
NOUT_M = GPW_A * CHUNK // SUPE_M // 2


@functools.cache
def _sc_edge_fn():
    return functools.partial(
        pl.kernel,
        mesh=_mesh(),
        compiler_params=pltpu.CompilerParams(use_tc_tiling_on_sc=False),
        out_type=[jax.ShapeDtypeStruct((EPAD, UB), bf16),
                  jax.ShapeDtypeStruct((N_NODES, UB), bf16)],
        scratch_types=[
            pltpu.VMEM((SUPE_M,), jnp.int32),
            pltpu.VMEM((SUPE_M,), jnp.int32),
            pltpu.VMEM((SUPE_M,), jnp.int32),
            pltpu.VMEM((SUPE_M,), jnp.int32),
            pltpu.VMEM((SUPE_M,), jnp.int32),
            pltpu.VMEM((SUPE_M,), jnp.int32),
            pltpu.VMEM((SUPE_M, 2 * UB), bf16),
            pltpu.VMEM((SUPE_M, 2 * UB), bf16),
            pltpu.VMEM((SUPE_M, UB), bf16),
            pltpu.VMEM((SUPE_M, UB), bf16),
            pltpu.VMEM((SUPE_M, UB), bf16),
            pltpu.VMEM((SUPE_M, UB), bf16),
            pltpu.VMEM((SUPE_M, UB), bf16),
            pltpu.VMEM((SUPE_M, UB), bf16),
            pltpu.VMEM_SHARED((ROWS_PAD, UB), bf16),
            pltpu.SemaphoreType.DMA,
            pltpu.SemaphoreType.DMA,
            pltpu.SemaphoreType.DMA,
            pltpu.SemaphoreType.DMA,
            pltpu.SemaphoreType.DMA,
            pltpu.SemaphoreType.DMA,
        ],
    )(_sc_edge_body)


def _sc_edge_body(src_h, dst_h, x24_h, x3_h, s_h, zeros_h, g_h, agg_h,
                  srcv0, srcv1, dstv0, dstv1, lidx0, lidx1,
                  r240, r241, r30, r31, sv0, sv1, ms0, ms1, aggS,
                  semi0, semi1, semg0, semg1, semw0, semw1):
    core = lax.axis_index("c")
    sub = lax.axis_index("s")
    base = sub * GPW_A * CHUNK
    srcv = (srcv0, srcv1)
    dstv = (dstv0, dstv1)
    lidx = (lidx0, lidx1)
    r24 = (r240, r241)
    r3 = (r30, r31)
    sv = (sv0, sv1)
    ms = (ms0, ms1)
    semi = (semi0, semi1)
    semg = (semg0, semg1)
    semw = (semw0, semw1)
    pltpu.sync_copy(zeros_h, aggS.at[pl.ds(sub * RPS, RPS)])
    plsc.subcore_barrier()

    def body(it, carry):
        offs = [pl.multiple_of(base + (it * 2 + b) * SUPE_M, SUPE_M)
                for b in range(2)]
        icps = []
        for b in range(2):
            icps.append(pltpu.async_copy(
                src_h.at[pl.ds(offs[b], SUPE_M)], srcv[b], semi[b]))
            icps.append(pltpu.async_copy(
                dst_h.at[pl.ds(offs[b], SUPE_M)], dstv[b], semi[b]))
            icps.append(pltpu.async_copy(
                s_h.at[pl.ds(offs[b], SUPE_M)], sv[b], semi[b]))
        x3cps = []
        for b in range(2):
            own = core == b

            @pl.when(own & (it > 0))
            def _(b=b):
                pltpu.make_async_copy(
                    r3[b], g_h.at[pl.ds(offs[b], SUPE_M)], semw[b]).wait()

            icps[3 * b].wait()
            icps[3 * b + 1].wait()
            icps[3 * b + 2].wait()
            gcp = pltpu.async_copy(x24_h.at[dstv[b]], r24[b], semg[b])
            x3cp = pltpu.make_async_copy(x3_h.at[srcv[b]], r3[b], semw[b])
            x3cps.append((gcp, x3cp, own))

            @pl.when(own)
            def _(x3cp=x3cp):
                x3cp.start()

            for kk in range(SUPE_M // 16):
                sl = pl.ds(kk * 16, 16)
                v = srcv[b][sl] - core * HALF
                ok = (v >= 0) & (v < HALF)
                lidx[b][sl] = jnp.where(ok, v, HALF)
        for b in range(2):
            gcp, x3cp, own = x3cps[b]
            gcp.wait()

            @pl.when(own)
            def _(x3cp=x3cp):
                x3cp.wait()

            def mul_row(i, c, b=b):
                for kk in range(UB // 32):
                    sl = pl.ds(kk * 32, 32)
                    ms[b][i, sl] = r24[b][i, sl] * sv[b][i, sl]
                return c

            lax.fori_loop(0, SUPE_M, mul_row, 0, unroll=4)

            @pl.when(own)
            def _(b=b):
                def g_row(i, c):
                    for kk in range(UB // 32):
                        sl = pl.ds(kk * 32, 32)
                        r3[b][i, sl] = (r3[b][i, sl]
                                        + r24[b][i, pl.ds(UB + kk * 32, 32)])
                    return c

                lax.fori_loop(0, SUPE_M, g_row, 0, unroll=4)
                pltpu.async_copy(
                    r3[b], g_h.at[pl.ds(offs[b], SUPE_M)], semw[b])

            pltpu.sync_copy(ms[b], aggS.at[lidx[b]], add=True)
        return carry

    lax.fori_loop(0, NOUT_M, body, 0)
    for b in range(2):
        @pl.when(core == b)
        def _(b=b):
            pltpu.make_async_copy(
                r3[b], g_h.at[pl.ds(base, SUPE_M)], semw[b]).wait()

    plsc.subcore_barrier()

    @pl.when(sub < NSUB - 1)
    def _():
        pltpu.sync_copy(
            aggS.at[pl.ds(sub * RPS, RPS)],
            agg_h.at[pl.ds(core * HALF + sub * RPS, RPS)],
        )

    @pl.when(sub == NSUB - 1)
    def _():
        pltpu.sync_copy(
            aggS.at[pl.ds((NSUB - 1) * RPS, RPS_LAST)],
            agg_h.at[pl.ds(core * HALF + (NSUB - 1) * RPS, RPS_LAST)],
        )


EBLK = 10000
EGRID = N_EDGES // EBLK


def _ninit_body(nodes, v0W, v0b, vRW, vRb, posW, R, cosb, sinb, maskb, alpha,
                xo):
    i = pl.program_id(0)
    nb = nodes[...]
    xb = jnp.dot(nb, v0W[...], preferred_element_type=f32) + v0b[...]

    @pl.when(i < MP // 5000)
    def _():
        place = jnp.dot(nb, vRW[...], preferred_element_type=f32) + vRb[...]
        p2 = jnp.dot(place, R[...], preferred_element_type=f32)
        pe = place * cosb[...] + p2 * sinb[...]
        pe = jnp.dot(pe, posW[...], preferred_element_type=f32) * alpha[0, 0]
        xo[...] = _silu(xb + pe * maskb[...])

    @pl.when(i >= MP // 5000)
    def _():
        xo[...] = _silu(xb)


def _node_init(nodes, v0W, v0b, vRW, vRb, posW, R, cosb, sinb, maskb, alpha):
    blk = 5000
    pspec = pl.BlockSpec((blk, UNITS), lambda i: (jnp.minimum(i, 1), 0))
    return pl.pallas_call(
        _ninit_body,
        grid=(N_NODES // blk,),
        in_specs=[
            pl.BlockSpec((blk, 2), lambda i: (i, 0)),
            pl.BlockSpec((2, UNITS), lambda i: (0, 0)),
            pl.BlockSpec((1, UNITS), lambda i: (0, 0)),
            pl.BlockSpec((2, UNITS), lambda i: (0, 0)),
            pl.BlockSpec((1, UNITS), lambda i: (0, 0)),
            pl.BlockSpec((UNITS, UNITS), lambda i: (0, 0)),
            pl.BlockSpec((UNITS, UNITS), lambda i: (0, 0)),
            pspec,
            pspec,
            pspec,
            pl.BlockSpec((1, 1), lambda i: (0, 0)),
        ],
        out_specs=pl.BlockSpec((blk, UNITS), lambda i: (i, 0)),
        out_shape=jax.ShapeDtypeStruct((N_NODES, UNITS), f32),
    )(nodes, v0W, v0b, vRW, vRb, posW, R, cosb, sinb, maskb, alpha)


def _einit_body(ea, e0W, e0b, w_ref, s_ref):
    z = ea[...] * e0W[...] + e0b[...]
    wv = _silu(z)
    w_ref[...] = wv
    zpad = jnp.zeros((wv.shape[0], UB - UNITS), dtype=bf16)
    s_ref[...] = jnp.concatenate([_sig(wv).astype(bf16), zpad], axis=1)


def _edge_init(ea, e0W, e0b):
    return pl.pallas_call(
        _einit_body,
        grid=(EGRID,),
        in_specs=[
            pl.BlockSpec((EBLK, 1), lambda i: (i, 0)),
            pl.BlockSpec((1, UNITS), lambda i: (0, 0)),
            pl.BlockSpec((1, UNITS), lambda i: (0, 0)),
        ],
        out_specs=[
            pl.BlockSpec((EBLK, UNITS), lambda i: (i, 0)),
            pl.BlockSpec((EBLK, UB), lambda i: (i, 0)),
        ],
        out_shape=[jax.ShapeDtypeStruct((N_EDGES, UNITS), f32),
                   jax.ShapeDtypeStruct((EPAD, UB), bf16)],
    )(ea, e0W, e0b)


NBLK = 5000
NGRID = N_NODES // NBLK


NBLK_M = 10000


def _nmm_body(x, Wc, bc, o1, o24, o3):
    acc = jnp.dot(x[...], Wc[...], preferred_element_type=f32) + bc[...]
    o1[...] = acc[:, 0 * UNITS:1 * UNITS]
    zpad = jnp.zeros((acc.shape[0], UB - UNITS), dtype=bf16)
    x2b = jnp.concatenate(
        [acc[:, 1 * UNITS:2 * UNITS].astype(bf16), zpad], axis=1)
    x4b = jnp.concatenate(
        [acc[:, 3 * UNITS:4 * UNITS].astype(bf16), zpad], axis=1)
    o24[...] = jnp.concatenate([x2b, x4b], axis=1)
    o3[...] = jnp.concatenate(
        [acc[:, 2 * UNITS:3 * UNITS].astype(bf16), zpad], axis=1)


def _node_mm4(x, Wc, bc):
    nb = jax.ShapeDtypeStruct((NPAD, UNITS), f32)
    nspec = pl.BlockSpec((NBLK_M, UNITS), lambda i: (i, 0))
    hspec = pl.BlockSpec((NBLK_M, UB), lambda i: (i, 0))
    return pl.pallas_call(
        _nmm_body,
        grid=(N_NODES // NBLK_M,),
        in_specs=[
            nspec,
            pl.BlockSpec((UNITS, 4 * UNITS), lambda i: (0, 0)),
            pl.BlockSpec((1, 4 * UNITS), lambda i: (0, 0)),
        ],
        out_specs=[nspec,
                   pl.BlockSpec((NBLK_M, 2 * UB), lambda i: (i, 0)), hspec],
        out_shape=[nb, jax.ShapeDtypeStruct((NPAD, 2 * UB), bf16),
                   jax.ShapeDtypeStruct((NPAD, UB), bf16)],
    )(x, Wc, bc)


def _emm_body(w0, g, Wr, br, t_ref, sums_ref):
    pid = pl.program_id(0)
    t = (jnp.dot(w0[...], Wr[...], preferred_element_type=f32) + br[...]
         + g[:, 0:UNITS].astype(f32))
    t_ref[...] = t

    @pl.when(pid == 0)
    def _():
        sums_ref[...] = jnp.zeros_like(sums_ref)

    sums_ref[0:1, :] += jnp.sum(t, axis=0, keepdims=True)
    sums_ref[1:2, :] += jnp.sum(t * t, axis=0, keepdims=True)


def _edge_mm_stats(w0, g, Wr, br):
    return pl.pallas_call(
        _emm_body,
        grid=(EGRID,),
        in_specs=[
            pl.BlockSpec((EBLK, UNITS), lambda i: (i, 0)),
            pl.BlockSpec((EBLK, UB), lambda i: (i, 0)),
            pl.BlockSpec((UNITS, UNITS), lambda i: (0, 0)),
            pl.BlockSpec((1, UNITS), lambda i: (0, 0)),
        ],
        out_specs=[
            pl.BlockSpec((EBLK, UNITS), lambda i: (i, 0)),
            pl.BlockSpec((2, UNITS), lambda i: (0, 0)),
        ],
        out_shape=[
            jax.ShapeDtypeStruct((N_EDGES, UNITS), f32),
            jax.ShapeDtypeStruct((2, UNITS), f32),
        ],
    )(w0, g, Wr, br)


def _nstat_body(x1, agg, cnt, t_ref, sums_ref):
    pid = pl.program_id(0)
    cn = cnt[...]
    ag = agg[:, 0:UNITS].astype(f32)
    a = jnp.where(cn > 0, ag / jnp.maximum(cn, 1.0), 0.0)
    t = x1[...] + a
    t_ref[...] = t

    @pl.when(pid == 0)
    def _():
        sums_ref[...] = jnp.zeros_like(sums_ref)

    sums_ref[0:1, :] += jnp.sum(t, axis=0, keepdims=True)
    sums_ref[1:2, :] += jnp.sum(t * t, axis=0, keepdims=True)


def _napply_body(t, x0, sums, gm, bt, xo):
    inv_n = 1.0 / N_NODES
    mu = sums[0:1, :] * inv_n
    var = sums[1:2, :] * inv_n - mu * mu
    y = (t[...] - mu) * lax.rsqrt(var + 1e-5) * gm[...] + bt[...]
    xo[...] = x0[...] + _silu(y)


def _node_update(x1, agg, cnt, x0, gm, bt):
    nb = jax.ShapeDtypeStruct((N_NODES, UNITS), f32)
    nspec = pl.BlockSpec((NBLK, UNITS), lambda i: (i, 0))
    mspec = pl.BlockSpec((NBLK_M, UNITS), lambda i: (i, 0))
    cspec = pl.BlockSpec((1, UNITS), lambda i: (0, 0))
    t, sums = pl.pallas_call(
        _nstat_body,
        grid=(N_NODES // NBLK_M,),
        in_specs=[mspec, pl.BlockSpec((NBLK_M, UB), lambda i: (i, 0)), mspec],
        out_specs=[mspec, pl.BlockSpec((2, UNITS), lambda i: (0, 0))],
        out_shape=[nb, jax.ShapeDtypeStruct((2, UNITS), f32)],
    )(x1, agg, cnt)
    return pl.pallas_call(
        _napply_body,
        grid=(NGRID,),
        in_specs=[nspec, nspec,
                  pl.BlockSpec((2, UNITS), lambda i: (0, 0)), cspec, cspec],
        out_specs=nspec,
        out_shape=nb,
    )(t, x0, sums, gm, bt)


def _eapply_body(t, w0, sums, gm, bt, w_ref, s_ref):
    inv_n = 1.0 / N_EDGES
    mu = sums[0:1, :] * inv_n
    var = sums[1:2, :] * inv_n - mu * mu
    y = (t[...] - mu) * lax.rsqrt(var + 1e-5) * gm[...] + bt[...]
    wn = w0[...] + _silu(y)
    w_ref[...] = wn
    zpad = jnp.zeros((wn.shape[0], UB - UNITS), dtype=bf16)
    s_ref[...] = jnp.concatenate([_sig(wn).astype(bf16), zpad], axis=1)


def _edge_apply(t, w0, sums, gm, bt):
    eb = jax.ShapeDtypeStruct((N_EDGES, UNITS), f32)
    sb = jax.ShapeDtypeStruct((EPAD, UB), bf16)
    return pl.pallas_call(
        _eapply_body,
        grid=(EGRID,),
        in_specs=[
            pl.BlockSpec((EBLK, UNITS), lambda i: (i, 0)),
            pl.BlockSpec((EBLK, UNITS), lambda i: (i, 0)),
            pl.BlockSpec((2, UNITS), lambda i: (0, 0)),
            pl.BlockSpec((1, UNITS), lambda i: (0, 0)),
            pl.BlockSpec((1, UNITS), lambda i: (0, 0)),
        ],
        out_specs=[
            pl.BlockSpec((EBLK, UNITS), lambda i: (i, 0)),
            pl.BlockSpec((EBLK, UB), lambda i: (i, 0)),
        ],
        out_shape=[eb, sb],
    )(t, w0, sums, gm, bt)


def kernel(nodes, edge_index, edge_attr, M, alpha, pos_W, v0_W, v0_b, vR_W,
           vR_b, e0_W, e0_b, V1_W, V1_b, V2_W, V2_b, V3_W, V3_b, V4_W, V4_b,
           Ew_W, Ew_b, VBN_g, VBN_b, EBN_g, EBN_b):
    src = edge_index[0]
    dst = edge_index[1]
    npad = EPAD - N_EDGES
    srcp = jnp.concatenate(
        [src, jnp.full((npad,), N_NODES, dtype=jnp.int32)])
    dstp = jnp.concatenate([dst, jnp.zeros((npad,), dtype=jnp.int32)])

    ids = jnp.arange(UNITS // 2, dtype=f32)
    theta = jnp.power(1000.0, -2.0 * ids / UNITS)
    emb = jnp.arange(MP, dtype=f32)[:, None] * theta
    cosb = jnp.repeat(jnp.sin(emb), 2, axis=-1)
    sinb = jnp.repeat(jnp.cos(emb), 2, axis=-1)
    maskb = jnp.broadcast_to(
        (jnp.arange(MP) < M)[:, None], (MP, UNITS)).astype(f32)
    k2 = jnp.arange(UNITS // 2)
    R = (jnp.zeros((UNITS, UNITS), f32)
         .at[2 * k2 + 1, 2 * k2].set(-1.0)
         .at[2 * k2, 2 * k2 + 1].set(1.0))

    ones_h = jnp.ones((SUPE_C, UNITS), f32)
    zeros_h = jnp.zeros((RPS, UNITS), f32)
    zeros_hb = jnp.zeros((RPS, UB), bf16)

    x = _node_init(nodes, v0_W, v0_b.reshape(1, -1), vR_W,
                   vR_b.reshape(1, -1), pos_W, R, cosb, sinb, maskb,
                   alpha.reshape(1, 1))
    w, s = _edge_init(edge_attr, e0_W, e0_b.reshape(1, -1))
    cntf = _sc_cnt_fn()(srcp, ones_h, zeros_h)

    for i in range(DEPTH):
        Wc = jnp.concatenate([V1_W[i], V2_W[i], V3_W[i], V4_W[i]], axis=1)
        bc = jnp.concatenate([V1_b[i], V2_b[i], V3_b[i], V4_b[i]]).reshape(1, -1)
        x1, x24, x3 = _node_mm4(x, Wc, bc)
        g, aggr = _sc_edge_fn()(srcp, dstp, x24, x3, s, zeros_hb)
        t, sums = _edge_mm_stats(w, g, Ew_W[i], Ew_b[i].reshape(1, -1))
        x = _node_update(x1, aggr, cntf, x, VBN_g[i].reshape(1, -1),
                         VBN_b[i].reshape(1, -1))
        w, s = _edge_apply(t, w, sums, EBN_g[i].reshape(1, -1),
                           EBN_b[i].reshape(1, -1))
    return (x, w)


# speedup vs baseline: 1.9235x; 1.0264x over previous
import functools

import jax
import jax.numpy as jnp
from jax import lax
from jax.experimental import pallas as pl
from jax.experimental.pallas import tpu as pltpu
from jax.experimental.pallas import tpu_sc as plsc

N_NODES = 50000
N_EDGES = 800000
UNITS = 48
DEPTH = 12
MP = 10000

NCORE = 2
NSUB = 16
CHUNK = 128
NCH = N_EDGES // CHUNK
HALF = N_NODES // 2
ROWS_PAD = 25008
RPS = ROWS_PAD // NSUB
RPS_LAST = HALF - RPS * (NSUB - 1)

EPAD = 819200
NPAD = 50008
SUPE = 512
GPW_G = EPAD // CHUNK // (NCORE * NSUB)
NOUT_G = GPW_G * CHUNK // SUPE // 2
GPW_A = EPAD // CHUNK // NSUB
SUPE_A = 256
NOUT_A = GPW_A * CHUNK // SUPE_A // 2

f32 = jnp.float32
bf16 = jnp.bfloat16
UB = 64


@functools.cache
def _mesh():
    return plsc.VectorSubcoreMesh(core_axis_name="c", subcore_axis_name="s")


def _sig(z):
    return 1.0 / (1.0 + jnp.exp(-z))


def _silu(z):
    return z * _sig(z)


SUPE_C = 512
NOUT_C = GPW_A * CHUNK // SUPE_C // 2


@functools.cache
def _sc_cnt_fn():
    return functools.partial(
        pl.kernel,
        mesh=_mesh(),
        compiler_params=pltpu.CompilerParams(use_tc_tiling_on_sc=False),
        out_type=jax.ShapeDtypeStruct((N_NODES, UNITS), f32),
        scratch_types=[
            pltpu.VMEM((SUPE_C,), jnp.int32),
            pltpu.VMEM((SUPE_C,), jnp.int32),
            pltpu.VMEM((SUPE_C,), jnp.int32),
            pltpu.VMEM((SUPE_C,), jnp.int32),
            pltpu.VMEM((SUPE_C, UNITS), f32),
            pltpu.VMEM_SHARED((ROWS_PAD, UNITS), f32),
            pltpu.SemaphoreType.DMA,
            pltpu.SemaphoreType.DMA,
        ],
    )(_sc_cnt_body)


def _sc_cnt_body(src_h, ones_h, zeros_h, out_h,
                 srcv0, srcv1, lidx0, lidx1, ones_v, aggS, semi0, semi1):
    core = lax.axis_index("c")
    sub = lax.axis_index("s")
    base = sub * GPW_A * CHUNK
    srcv = (srcv0, srcv1)
    lidx = (lidx0, lidx1)
    semi = (semi0, semi1)
    pltpu.sync_copy(zeros_h, aggS.at[pl.ds(sub * RPS, RPS)])
    pltpu.sync_copy(ones_h, ones_v)
    plsc.subcore_barrier()

    def body(it, carry):
        offs = [pl.multiple_of(base + (it * 2 + b) * SUPE_C, SUPE_C)
                for b in range(2)]
        icps = [pltpu.async_copy(
            src_h.at[pl.ds(offs[b], SUPE_C)], srcv[b], semi[b])
            for b in range(2)]
        for b in range(2):
            icps[b].wait()
            for kk in range(SUPE_C // 16):
                sl = pl.ds(kk * 16, 16)
                v = srcv[b][sl] - core * HALF
                ok = (v >= 0) & (v < HALF)
                lidx[b][sl] = jnp.where(ok, v, HALF)
            pltpu.sync_copy(ones_v, aggS.at[lidx[b]], add=True)
        return carry

    lax.fori_loop(0, NOUT_C, body, 0)
    plsc.subcore_barrier()

    @pl.when(sub < NSUB - 1)
    def _():
        pltpu.sync_copy(
            aggS.at[pl.ds(sub * RPS, RPS)],
            out_h.at[pl.ds(core * HALF + sub * RPS, RPS)],
        )

    @pl.when(sub == NSUB - 1)
    def _():
        pltpu.sync_copy(
            aggS.at[pl.ds((NSUB - 1) * RPS, RPS_LAST)],
            out_h.at[pl.ds(core * HALF + (NSUB - 1) * RPS, RPS_LAST)],
        )


SUPE_M = 256
NOUT_M = GPW_A * CHUNK // SUPE_M // 2


@functools.cache
def _sc_edge_fn():
    return functools.partial(
        pl.kernel,
        mesh=_mesh(),
        compiler_params=pltpu.CompilerParams(use_tc_tiling_on_sc=False),
        out_type=[jax.ShapeDtypeStruct((EPAD, UB), bf16),
                  jax.ShapeDtypeStruct((N_NODES, UB), bf16)],
        scratch_types=[
            pltpu.VMEM((SUPE_M,), jnp.int32),
            pltpu.VMEM((SUPE_M,), jnp.int32),
            pltpu.VMEM((SUPE_M,), jnp.int32),
            pltpu.VMEM((SUPE_M,), jnp.int32),
            pltpu.VMEM((SUPE_M,), jnp.int32),
            pltpu.VMEM((SUPE_M,), jnp.int32),
            pltpu.VMEM((SUPE_M, 2 * UB), bf16),
            pltpu.VMEM((SUPE_M, 2 * UB), bf16),
            pltpu.VMEM((SUPE_M, UB), bf16),
            pltpu.VMEM((SUPE_M, UB), bf16),
            pltpu.VMEM((SUPE_M, UB), bf16),
            pltpu.VMEM((SUPE_M, UB), bf16),
            pltpu.VMEM((SUPE_M, UB), bf16),
            pltpu.VMEM_SHARED((ROWS_PAD, UB), bf16),
            pltpu.SemaphoreType.DMA,
            pltpu.SemaphoreType.DMA,
            pltpu.SemaphoreType.DMA,
            pltpu.SemaphoreType.DMA,
            pltpu.SemaphoreType.DMA,
            pltpu.SemaphoreType.DMA,
        ],
    )(_sc_edge_body)


def _sc_edge_body(src_h, dst_h, x24_h, x3_h, s_h, zeros_h, g_h, agg_h,
                  srcv0, srcv1, dstv0, dstv1, lidx0, lidx1,
                  r240, r241, r30, r31, sv0, sv1, ms0, aggS,
                  semi0, semi1, semg0, semg1, semw0, semw1):
    core = lax.axis_index("c")
    sub = lax.axis_index("s")
    base = sub * GPW_A * CHUNK
    srcv = (srcv0, srcv1)
    dstv = (dstv0, dstv1)
    lidx = (lidx0, lidx1)
    r24 = (r240, r241)
    r3 = (r30, r31)
    sv = (sv0, sv1)
    ms = (ms0, ms0)
    semi = (semi0, semi1)
    semg = (semg0, semg1)
    semw = (semw0, semw1)
    pltpu.sync_copy(zeros_h, aggS.at[pl.ds(sub * RPS, RPS)])
    plsc.subcore_barrier()

    def body(it, carry):
        offs = [pl.multiple_of(base + (it * 2 + b) * SUPE_M, SUPE_M)
                for b in range(2)]
        icps = []
        for b in range(2):
            icps.append(pltpu.async_copy(
                src_h.at[pl.ds(offs[b], SUPE_M)], srcv[b], semi[b]))
            icps.append(pltpu.async_copy(
                dst_h.at[pl.ds(offs[b], SUPE_M)], dstv[b], semi[b]))
            icps.append(pltpu.async_copy(
                s_h.at[pl.ds(offs[b], SUPE_M)], sv[b], semi[b]))
        x3cps = []
        for b in range(2):
            own = core == b

            @pl.when(own & (it > 0))
            def _(b=b):
                pltpu.make_async_copy(
                    r3[b], g_h.at[pl.ds(offs[b], SUPE_M)], semw[b]).wait()

            icps[3 * b].wait()
            icps[3 * b + 1].wait()
            icps[3 * b + 2].wait()
            gcp = pltpu.async_copy(x24_h.at[dstv[b]], r24[b], semg[b])
            x3cp = pltpu.make_async_copy(x3_h.at[srcv[b]], r3[b], semw[b])
            x3cps.append((gcp, x3cp, own))

            @pl.when(own)
            def _(x3cp=x3cp):
                x3cp.start()

            for kk in range(SUPE_M // 16):
                sl = pl.ds(kk * 16, 16)
                v = srcv[b][sl] - core * HALF
                ok = (v >= 0) & (v < HALF)
                lidx[b][sl] = jnp.where(ok, v, HALF)
        for b in range(2):
            gcp, x3cp, own = x3cps[b]
            gcp.wait()

            @pl.when(own)
            def _(x3cp=x3cp):
                x3cp.wait()

            def mul_row(i, c, b=b):
                for kk in range(UB // 32):
                    sl = pl.ds(kk * 32, 32)
                    ms[b][i, sl] = r24[b][i, sl] * sv[b][i, sl]
                return c

            lax.fori_loop(0, SUPE_M, mul_row, 0, unroll=4)

            @pl.when(own)
            def _(b=b):
                def g_row(i, c):
                    for kk in range(UB // 32):
                        sl = pl.ds(kk * 32, 32)
                        r3[b][i, sl] = (r3[b][i, sl]
                                        + r24[b][i, pl.ds(UB + kk * 32, 32)])
                    return c

                lax.fori_loop(0, SUPE_M, g_row, 0, unroll=4)
                pltpu.async_copy(
                    r3[b], g_h.at[pl.ds(offs[b], SUPE_M)], semw[b])

            pltpu.sync_copy(ms[b], aggS.at[lidx[b]], add=True)
        return carry

    lax.fori_loop(0, NOUT_M, body, 0)
    for b in range(2):
        @pl.when(core == b)
        def _(b=b):
            pltpu.make_async_copy(
                r3[b], g_h.at[pl.ds(base, SUPE_M)], semw[b]).wait()

    plsc.subcore_barrier()

    @pl.when(sub < NSUB - 1)
    def _():
        pltpu.sync_copy(
            aggS.at[pl.ds(sub * RPS, RPS)],
            agg_h.at[pl.ds(core * HALF + sub * RPS, RPS)],
        )

    @pl.when(sub == NSUB - 1)
    def _():
        pltpu.sync_copy(
            aggS.at[pl.ds((NSUB - 1) * RPS, RPS_LAST)],
            agg_h.at[pl.ds(core * HALF + (NSUB - 1) * RPS, RPS_LAST)],
        )


EBLK = 10000
EGRID = N_EDGES // EBLK


def _ninit_body(nodes, v0W, v0b, vRW, vRb, posW, R, cosb, sinb, maskb, alpha,
                xo):
    i = pl.program_id(0)
    nb = nodes[...]
    xb = jnp.dot(nb, v0W[...], preferred_element_type=f32) + v0b[...]

    @pl.when(i < MP // 5000)
    def _():
        place = jnp.dot(nb, vRW[...], preferred_element_type=f32) + vRb[...]
        p2 = jnp.dot(place, R[...], preferred_element_type=f32)
        pe = place * cosb[...] + p2 * sinb[...]
        pe = jnp.dot(pe, posW[...], preferred_element_type=f32) * alpha[0, 0]
        xo[...] = _silu(xb + pe * maskb[...])

    @pl.when(i >= MP // 5000)
    def _():
        xo[...] = _silu(xb)


def _node_init(nodes, v0W, v0b, vRW, vRb, posW, R, cosb, sinb, maskb, alpha):
    blk = 5000
    pspec = pl.BlockSpec((blk, UNITS), lambda i: (jnp.minimum(i, 1), 0))
    return pl.pallas_call(
        _ninit_body,
        grid=(N_NODES // blk,),
        in_specs=[
            pl.BlockSpec((blk, 2), lambda i: (i, 0)),
            pl.BlockSpec((2, UNITS), lambda i: (0, 0)),
            pl.BlockSpec((1, UNITS), lambda i: (0, 0)),
            pl.BlockSpec((2, UNITS), lambda i: (0, 0)),
            pl.BlockSpec((1, UNITS), lambda i: (0, 0)),
            pl.BlockSpec((UNITS, UNITS), lambda i: (0, 0)),
            pl.BlockSpec((UNITS, UNITS), lambda i: (0, 0)),
            pspec,
            pspec,
            pspec,
            pl.BlockSpec((1, 1), lambda i: (0, 0)),
        ],
        out_specs=pl.BlockSpec((blk, UNITS), lambda i: (i, 0)),
        out_shape=jax.ShapeDtypeStruct((N_NODES, UNITS), f32),
    )(nodes, v0W, v0b, vRW, vRb, posW, R, cosb, sinb, maskb, alpha)


def _einit_body(ea, e0W, e0b, w_ref, s_ref):
    z = ea[...] * e0W[...] + e0b[...]
    wv = _silu(z)
    w_ref[...] = wv
    zpad = jnp.zeros((wv.shape[0], UB - UNITS), dtype=bf16)
    s_ref[...] = jnp.concatenate([_sig(wv).astype(bf16), zpad], axis=1)


def _edge_init(ea, e0W, e0b):
    return pl.pallas_call(
        _einit_body,
        grid=(EGRID,),
        in_specs=[
            pl.BlockSpec((EBLK, 1), lambda i: (i, 0)),
            pl.BlockSpec((1, UNITS), lambda i: (0, 0)),
            pl.BlockSpec((1, UNITS), lambda i: (0, 0)),
        ],
        out_specs=[
            pl.BlockSpec((EBLK, UNITS), lambda i: (i, 0)),
            pl.BlockSpec((EBLK, UB), lambda i: (i, 0)),
        ],
        out_shape=[jax.ShapeDtypeStruct((N_EDGES, UNITS), f32),
                   jax.ShapeDtypeStruct((EPAD, UB), bf16)],
    )(ea, e0W, e0b)


NBLK = 5000
NGRID = N_NODES // NBLK


NBLK_M = 10000


def _nmm_body(x, Wc, bc, o1, o24, o3):
    acc = jnp.dot(x[...], Wc[...], preferred_element_type=f32) + bc[...]
    o1[...] = acc[:, 0 * UNITS:1 * UNITS]
    zpad = jnp.zeros((acc.shape[0], UB - UNITS), dtype=bf16)
    x2b = jnp.concatenate(
        [acc[:, 1 * UNITS:2 * UNITS].astype(bf16), zpad], axis=1)
    x4b = jnp.concatenate(
        [acc[:, 3 * UNITS:4 * UNITS].astype(bf16), zpad], axis=1)
    o24[...] = jnp.concatenate([x2b, x4b], axis=1)
    o3[...] = jnp.concatenate(
        [acc[:, 2 * UNITS:3 * UNITS].astype(bf16), zpad], axis=1)


def _node_mm4(x, Wc, bc):
    nb = jax.ShapeDtypeStruct((NPAD, UNITS), f32)
    nspec = pl.BlockSpec((NBLK_M, UNITS), lambda i: (i, 0))
    hspec = pl.BlockSpec((NBLK_M, UB), lambda i: (i, 0))
    return pl.pallas_call(
        _nmm_body,
        grid=(N_NODES // NBLK_M,),
        in_specs=[
            nspec,
            pl.BlockSpec((UNITS, 4 * UNITS), lambda i: (0, 0)),
            pl.BlockSpec((1, 4 * UNITS), lambda i: (0, 0)),
        ],
        out_specs=[nspec,
                   pl.BlockSpec((NBLK_M, 2 * UB), lambda i: (i, 0)), hspec],
        out_shape=[nb, jax.ShapeDtypeStruct((NPAD, 2 * UB), bf16),
                   jax.ShapeDtypeStruct((NPAD, UB), bf16)],
    )(x, Wc, bc)


def _emm_body(w0, g, Wr, br, t_ref, sums_ref):
    pid = pl.program_id(0)
    t = (jnp.dot(w0[...], Wr[...], preferred_element_type=f32) + br[...]
         + g[:, 0:UNITS].astype(f32))
    t_ref[...] = t

    @pl.when(pid == 0)
    def _():
        sums_ref[...] = jnp.zeros_like(sums_ref)

    sums_ref[0:1, :] += jnp.sum(t, axis=0, keepdims=True)
    sums_ref[1:2, :] += jnp.sum(t * t, axis=0, keepdims=True)


def _edge_mm_stats(w0, g, Wr, br):
    return pl.pallas_call(
        _emm_body,
        grid=(EGRID,),
        in_specs=[
            pl.BlockSpec((EBLK, UNITS), lambda i: (i, 0)),
            pl.BlockSpec((EBLK, UB), lambda i: (i, 0)),
            pl.BlockSpec((UNITS, UNITS), lambda i: (0, 0)),
            pl.BlockSpec((1, UNITS), lambda i: (0, 0)),
        ],
        out_specs=[
            pl.BlockSpec((EBLK, UNITS), lambda i: (i, 0)),
            pl.BlockSpec((2, UNITS), lambda i: (0, 0)),
        ],
        out_shape=[
            jax.ShapeDtypeStruct((N_EDGES, UNITS), f32),
            jax.ShapeDtypeStruct((2, UNITS), f32),
        ],
    )(w0, g, Wr, br)


def _nstat_body(x1, agg, cnt, t_ref, sums_ref):
    pid = pl.program_id(0)
    cn = cnt[...]
    ag = agg[:, 0:UNITS].astype(f32)
    a = jnp.where(cn > 0, ag / jnp.maximum(cn, 1.0), 0.0)
    t = x1[...] + a
    t_ref[...] = t

    @pl.when(pid == 0)
    def _():
        sums_ref[...] = jnp.zeros_like(sums_ref)

    sums_ref[0:1, :] += jnp.sum(t, axis=0, keepdims=True)
    sums_ref[1:2, :] += jnp.sum(t * t, axis=0, keepdims=True)


def _napply_body(t, x0, sums, gm, bt, xo):
    inv_n = 1.0 / N_NODES
    mu = sums[0:1, :] * inv_n
    var = sums[1:2, :] * inv_n - mu * mu
    y = (t[...] - mu) * lax.rsqrt(var + 1e-5) * gm[...] + bt[...]
    xo[...] = x0[...] + _silu(y)


def _node_update(x1, agg, cnt, x0, gm, bt):
    nb = jax.ShapeDtypeStruct((N_NODES, UNITS), f32)
    nspec = pl.BlockSpec((NBLK, UNITS), lambda i: (i, 0))
    mspec = pl.BlockSpec((NBLK_M, UNITS), lambda i: (i, 0))
    cspec = pl.BlockSpec((1, UNITS), lambda i: (0, 0))
    t, sums = pl.pallas_call(
        _nstat_body,
        grid=(N_NODES // NBLK_M,),
        in_specs=[mspec, pl.BlockSpec((NBLK_M, UB), lambda i: (i, 0)), mspec],
        out_specs=[mspec, pl.BlockSpec((2, UNITS), lambda i: (0, 0))],
        out_shape=[nb, jax.ShapeDtypeStruct((2, UNITS), f32)],
    )(x1, agg, cnt)
    return pl.pallas_call(
        _napply_body,
        grid=(NGRID,),
        in_specs=[nspec, nspec,
                  pl.BlockSpec((2, UNITS), lambda i: (0, 0)), cspec, cspec],
        out_specs=nspec,
        out_shape=nb,
    )(t, x0, sums, gm, bt)


def _eapply_body(t, w0, sums, gm, bt, w_ref, s_ref):
    inv_n = 1.0 / N_EDGES
    mu = sums[0:1, :] * inv_n
    var = sums[1:2, :] * inv_n - mu * mu
    y = (t[...] - mu) * lax.rsqrt(var + 1e-5) * gm[...] + bt[...]
    wn = w0[...] + _silu(y)
    w_ref[...] = wn
    zpad = jnp.zeros((wn.shape[0], UB - UNITS), dtype=bf16)
    s_ref[...] = jnp.concatenate([_sig(wn).astype(bf16), zpad], axis=1)


def _edge_apply(t, w0, sums, gm, bt):
    eb = jax.ShapeDtypeStruct((N_EDGES, UNITS), f32)
    sb = jax.ShapeDtypeStruct((EPAD, UB), bf16)
    return pl.pallas_call(
        _eapply_body,
        grid=(EGRID,),
        in_specs=[
            pl.BlockSpec((EBLK, UNITS), lambda i: (i, 0)),
            pl.BlockSpec((EBLK, UNITS), lambda i: (i, 0)),
            pl.BlockSpec((2, UNITS), lambda i: (0, 0)),
            pl.BlockSpec((1, UNITS), lambda i: (0, 0)),
            pl.BlockSpec((1, UNITS), lambda i: (0, 0)),
        ],
        out_specs=[
            pl.BlockSpec((EBLK, UNITS), lambda i: (i, 0)),
            pl.BlockSpec((EBLK, UB), lambda i: (i, 0)),
        ],
        out_shape=[eb, sb],
    )(t, w0, sums, gm, bt)


def kernel(nodes, edge_index, edge_attr, M, alpha, pos_W, v0_W, v0_b, vR_W,
           vR_b, e0_W, e0_b, V1_W, V1_b, V2_W, V2_b, V3_W, V3_b, V4_W, V4_b,
           Ew_W, Ew_b, VBN_g, VBN_b, EBN_g, EBN_b):
    src = edge_index[0]
    dst = edge_index[1]
    npad = EPAD - N_EDGES
    srcp = jnp.concatenate(
        [src, jnp.full((npad,), N_NODES, dtype=jnp.int32)])
    dstp = jnp.concatenate([dst, jnp.zeros((npad,), dtype=jnp.int32)])

    ids = jnp.arange(UNITS // 2, dtype=f32)
    theta = jnp.power(1000.0, -2.0 * ids / UNITS)
    emb = jnp.arange(MP, dtype=f32)[:, None] * theta
    cosb = jnp.repeat(jnp.sin(emb), 2, axis=-1)
    sinb = jnp.repeat(jnp.cos(emb), 2, axis=-1)
    maskb = jnp.broadcast_to(
        (jnp.arange(MP) < M)[:, None], (MP, UNITS)).astype(f32)
    k2 = jnp.arange(UNITS // 2)
    R = (jnp.zeros((UNITS, UNITS), f32)
         .at[2 * k2 + 1, 2 * k2].set(-1.0)
         .at[2 * k2, 2 * k2 + 1].set(1.0))

    ones_h = jnp.ones((SUPE_C, UNITS), f32)
    zeros_h = jnp.zeros((RPS, UNITS), f32)
    zeros_hb = jnp.zeros((RPS, UB), bf16)

    x = _node_init(nodes, v0_W, v0_b.reshape(1, -1), vR_W,
                   vR_b.reshape(1, -1), pos_W, R, cosb, sinb, maskb,
                   alpha.reshape(1, 1))
    w, s = _edge_init(edge_attr, e0_W, e0_b.reshape(1, -1))
    cntf = _sc_cnt_fn()(srcp, ones_h, zeros_h)

    for i in range(DEPTH):
        Wc = jnp.concatenate([V1_W[i], V2_W[i], V3_W[i], V4_W[i]], axis=1)
        bc = jnp.concatenate([V1_b[i], V2_b[i], V3_b[i], V4_b[i]]).reshape(1, -1)
        x1, x24, x3 = _node_mm4(x, Wc, bc)
        g, aggr = _sc_edge_fn()(srcp, dstp, x24, x3, s, zeros_hb)
        t, sums = _edge_mm_stats(w, g, Ew_W[i], Ew_b[i].reshape(1, -1))
        x = _node_update(x1, aggr, cntf, x, VBN_g[i].reshape(1, -1),
                         VBN_b[i].reshape(1, -1))
        w, s = _edge_apply(t, w, sums, EBN_g[i].reshape(1, -1),
                           EBN_b[i].reshape(1, -1))
    return (x, w)
